# Initial kernel scaffold; baseline (speedup 1.0000x reference)
import functools
import math

import jax
import jax.numpy as jnp
from jax import lax
from jax.experimental import pallas as pl
from jax.experimental.pallas import tpu as pltpu
from jax.experimental.pallas import tpu_sc as plsc

D = 128
N = 50000
E_RAW = 600000
LANES = 16
NSUB = 16

BLKE = 1024
NBLK = 37
EPAD = BLKE * NSUB * NBLK
FLUSH = 128
CAP = 160
REG = 12512
NPAD = 4 * REG
ACC_ROWS = 12528
DUMMY = REG
SENTINEL = 1 << 30
SUB_ROWS = REG // NSUB
CNT_CHUNK = 784
CNT_LAST = REG - 15 * CNT_CHUNK

_BN_SCALE = 1.0 / math.sqrt(1.0 + 1e-5)


def _segsum_body(x_hbm, src_hbm, dst_hbm, sum_hbm, cnt_hbm,
                 rows_v, blkb_src, blkb_dst, src_buf, dst_buf,
                 src_dma, dst_dma, zbuf, ones_v, off_ref, acc, cnt_sh, sem):
    c = lax.axis_index("c")
    s = lax.axis_index("s")
    z16f = jnp.zeros((LANES,), jnp.float32)
    z16i = jnp.zeros((LANES,), jnp.int32)
    one16 = jnp.ones((LANES,), jnp.float32)
    row0 = s * SUB_ROWS

    @pl.loop(0, FLUSH)
    def _(r):
        for k in range(D // LANES):
            rows_v[r, pl.ds(k * LANES, LANES)] = z16f

    @pl.loop(0, CNT_CHUNK // LANES)
    def _(i):
        zbuf[pl.ds(i * LANES, LANES)] = z16f

    for k in range(FLUSH // LANES):
        ones_v[pl.ds(k * LANES, LANES)] = one16

    for p in range(2):
        base = (2 * p + c) * REG

        for k in range(SUB_ROWS // FLUSH):
            pltpu.sync_copy(rows_v, acc.at[pl.ds(row0 + k * FLUSH, FLUSH)])
        pltpu.sync_copy(rows_v.at[pl.ds(0, SUB_ROWS % FLUSH)],
                        acc.at[pl.ds(row0 + (SUB_ROWS // FLUSH) * FLUSH,
                                     SUB_ROWS % FLUSH)])

        @pl.when(s < NSUB - 1)
        def _():
            pltpu.sync_copy(zbuf, cnt_sh.at[pl.ds(s * CNT_CHUNK, CNT_CHUNK)])

        @pl.when(s == NSUB - 1)
        def _():
            pltpu.sync_copy(zbuf.at[pl.ds(0, CNT_LAST)],
                            cnt_sh.at[pl.ds(s * CNT_CHUNK, CNT_LAST)])

        for k in range(CAP // LANES):
            src_buf[pl.ds(k * LANES, LANES)] = z16i
            dst_buf[pl.ds(k * LANES, LANES)] = z16i
        off_ref[0] = 0
        plsc.subcore_barrier()

        def _flush():
            for k in range(FLUSH // LANES):
                sl = pl.ds(k * LANES, LANES)
                src_dma[sl] = src_buf[sl]
                dst_dma[sl] = dst_buf[sl]
            pltpu.async_copy(x_hbm.at[src_dma], rows_v, sem).wait()
            pltpu.sync_copy(rows_v, acc.at[dst_dma], add=True)
            pltpu.sync_copy(ones_v, cnt_sh.at[dst_dma], add=True)

        @pl.loop(0, NBLK)
        def _(b):
            e0 = (b * NSUB + s) * BLKE
            pltpu.sync_copy(src_hbm.at[pl.ds(e0, BLKE)], blkb_src)
            pltpu.sync_copy(dst_hbm.at[pl.ds(e0, BLKE)], blkb_dst)

            @pl.loop(0, BLKE // LANES)
            def _(k):
                sl = pl.ds(k * LANES, LANES)
                d16 = blkb_dst[sl]
                s16 = blkb_src[sl]
                m = (d16 >= base) & (d16 < base + REG)
                mi = m.astype(jnp.int32)
                off0 = off_ref[0]
                pos = jnp.cumsum(mi) - mi + off0
                plsc.store_scatter(src_buf, [pos], s16, m)
                plsc.store_scatter(dst_buf, [pos], d16 - base, m)
                off1 = off0 + jnp.sum(mi)
                off_ref[0] = off1

                @pl.when(off1 >= FLUSH)
                def _():
                    _flush()
                    src_buf[pl.ds(0, LANES)] = src_buf[pl.ds(FLUSH, LANES)]
                    dst_buf[pl.ds(0, LANES)] = dst_buf[pl.ds(FLUSH, LANES)]
                    off_ref[0] = off1 - FLUSH

        offf = off_ref[0]

        @pl.when(offf > 0)
        def _():
            for k in range(FLUSH // LANES):
                sl = pl.ds(k * LANES, LANES)
                posv = lax.iota(jnp.int32, LANES) + (k * LANES)
                keep = posv < offf
                src_dma[sl] = src_buf[sl]
                dst_dma[sl] = jnp.where(
                    keep, dst_buf[sl], jnp.full((LANES,), DUMMY, jnp.int32))
            pltpu.async_copy(x_hbm.at[src_dma], rows_v, sem).wait()
            pltpu.sync_copy(rows_v, acc.at[dst_dma], add=True)
            pltpu.sync_copy(ones_v, cnt_sh.at[dst_dma], add=True)

        plsc.subcore_barrier()

        pltpu.sync_copy(acc.at[pl.ds(row0, SUB_ROWS)],
                        sum_hbm.at[pl.ds(base + row0, SUB_ROWS)])

        @pl.when(s < NSUB - 1)
        def _():
            pltpu.sync_copy(cnt_sh.at[pl.ds(s * CNT_CHUNK, CNT_CHUNK)],
                            cnt_hbm.at[pl.ds(base + s * CNT_CHUNK, CNT_CHUNK)])

        @pl.when(s == NSUB - 1)
        def _():
            pltpu.sync_copy(cnt_sh.at[pl.ds(s * CNT_CHUNK, CNT_LAST)],
                            cnt_hbm.at[pl.ds(base + s * CNT_CHUNK, CNT_LAST)])

        plsc.subcore_barrier()


def _sc_segsum(x, src, dst):
    mesh = plsc.VectorSubcoreMesh(core_axis_name="c", subcore_axis_name="s")
    f = pl.kernel(
        _segsum_body,
        out_type=(jax.ShapeDtypeStruct((NPAD, D), jnp.float32),
                  jax.ShapeDtypeStruct((NPAD,), jnp.float32)),
        mesh=mesh,
        scratch_types=[
            pltpu.VMEM((FLUSH, D), jnp.float32),
            pltpu.VMEM((BLKE,), jnp.int32),
            pltpu.VMEM((BLKE,), jnp.int32),
            pltpu.VMEM((CAP,), jnp.int32),
            pltpu.VMEM((CAP,), jnp.int32),
            pltpu.VMEM((FLUSH,), jnp.int32),
            pltpu.VMEM((FLUSH,), jnp.int32),
            pltpu.VMEM((CNT_CHUNK,), jnp.float32),
            pltpu.VMEM((FLUSH,), jnp.float32),
            pltpu.SMEM((1,), jnp.int32),
            pltpu.VMEM_SHARED((ACC_ROWS, D), jnp.float32),
            pltpu.VMEM_SHARED((ACC_ROWS,), jnp.float32),
            pltpu.SemaphoreType.DMA,
        ],
    )
    return f(x, src, dst)


def _dense_body(sum_ref, cnt_ref, x_ref, wl_ref, wr_ref, b_ref, o_ref):
    rcp = 1.0 / jnp.maximum(cnt_ref[...], 1.0)
    agg = sum_ref[...] * rcp
    y = lax.dot_general(agg, wl_ref[...], (((1,), (0,)), ((), ())),
                        precision=lax.Precision.HIGHEST,
                        preferred_element_type=jnp.float32)
    y = y + lax.dot_general(x_ref[...], wr_ref[...], (((1,), (0,)), ((), ())),
                            precision=lax.Precision.HIGHEST,
                            preferred_element_type=jnp.float32)
    o_ref[...] = jnp.maximum(y + b_ref[...], 0.0)


_BLKR = 1000


def _tc_dense(sum_pad, cnt_pad, x_dst, Wl, bl, Wr, gamma, beta):
    scale = gamma * _BN_SCALE
    wl = Wl.T * scale[None, :]
    wr = Wr.T * scale[None, :]
    bb = (bl * scale + beta).reshape(1, D)
    cnt2 = cnt_pad.reshape(-1, 1)
    return pl.pallas_call(
        _dense_body,
        grid=(N // _BLKR,),
        in_specs=[
            pl.BlockSpec((_BLKR, D), lambda i: (i, 0)),
            pl.BlockSpec((_BLKR, 1), lambda i: (i, 0)),
            pl.BlockSpec((_BLKR, D), lambda i: (i, 0)),
            pl.BlockSpec((D, D), lambda i: (0, 0)),
            pl.BlockSpec((D, D), lambda i: (0, 0)),
            pl.BlockSpec((1, D), lambda i: (0, 0)),
        ],
        out_specs=pl.BlockSpec((_BLKR, D), lambda i: (i, 0)),
        out_shape=jax.ShapeDtypeStruct((N, D), jnp.float32),
    )(sum_pad, cnt2, x_dst, wl, wr, bb)


def _pad_edges(edge):
    src = jnp.concatenate(
        [edge[0].astype(jnp.int32), jnp.zeros((EPAD - E_RAW,), jnp.int32)])
    dst = jnp.concatenate(
        [edge[1].astype(jnp.int32),
         jnp.full((EPAD - E_RAW,), SENTINEL, jnp.int32)])
    return src, dst


def kernel(x_user, x_event, edge_e2u, edge_u2e,
           Wl_u0, bl_u0, Wr_u0, gamma_u0, beta_u0,
           Wl_e0, bl_e0, Wr_e0, gamma_e0, beta_e0,
           Wl_u1, bl_u1, Wr_u1, gamma_u1, beta_u1,
           Wl_e1, bl_e1, Wr_e1, gamma_e1, beta_e1):
    se2u, de2u = _pad_edges(edge_e2u)
    su2e, du2e = _pad_edges(edge_u2e)
    params = {
        "u": [(Wl_u0, bl_u0, Wr_u0, gamma_u0, beta_u0),
              (Wl_u1, bl_u1, Wr_u1, gamma_u1, beta_u1)],
        "e": [(Wl_e0, bl_e0, Wr_e0, gamma_e0, beta_e0),
              (Wl_e1, bl_e1, Wr_e1, gamma_e1, beta_e1)],
    }
    xu, xe = x_user, x_event
    for i in range(2):
        su, cu = _sc_segsum(xe, se2u, de2u)
        xu = _tc_dense(su, cu, xu, *params["u"][i])
        se, ce = _sc_segsum(xu, su2e, du2e)
        xe = _tc_dense(se, ce, xe, *params["e"][i])
    return (xu, xe)


# speedup vs baseline: 5.4388x; 5.4388x over previous
import dataclasses
import functools
import math

import jax
import jax.numpy as jnp
from jax import lax
from jax.experimental import pallas as pl
from jax.experimental.pallas import tpu as pltpu
from jax.experimental.pallas import tpu_sc as plsc

D = 128
N = 50000
E_RAW = 600000
LANES = 16
NSUB = 16

BLKE = 1024
NBLK = 37
EPAD = BLKE * NSUB * NBLK
FLUSH = 128
CAP = 160
REG = 12512
NPAD = 4 * REG
ACC_ROWS = 12528
DUMMY = REG
SENTINEL = 1 << 30
ROW_CHUNK = 784
ROW_LAST = REG - 15 * ROW_CHUNK
CNT_CHUNK = 784
CNT_LAST = REG - 15 * CNT_CHUNK

_BN_SCALE = 1.0 / math.sqrt(1.0 + 1e-5)


def _segsum_body(x_hbm, src_hbm, dst_hbm, sum_hbm, cnt_hbm,
                 rows_v, blkb_src, blkb_dst, src_buf, dst_buf,
                 src_dma, dst_dma, zbuf, ones_v, cbuf, off_ref, acc, cnt_sh,
                 sem):
    c = lax.axis_index("c")
    s = lax.axis_index("s")
    z16f = jnp.zeros((LANES,), jnp.float32)
    z16i = jnp.zeros((LANES,), jnp.int32)
    one16 = jnp.ones((LANES,), jnp.float32)
    row0 = s * ROW_CHUNK

    @pl.loop(0, CNT_CHUNK // LANES)
    def _(i):
        zbuf[pl.ds(i * LANES, LANES)] = z16f

    for k in range(FLUSH // LANES):
        ones_v[pl.ds(k * LANES, LANES)] = one16

    for p in range(2):
        base = (2 * p + c) * REG

        @pl.loop(0, FLUSH)
        def _(r):
            for k in range(D // LANES):
                rows_v[r, pl.ds(k * LANES, LANES)] = z16f

        for k in range(5):
            pltpu.sync_copy(rows_v, acc.at[pl.ds(row0 + k * FLUSH, FLUSH)])

        @pl.when(s < NSUB - 1)
        def _():
            pltpu.sync_copy(rows_v, acc.at[pl.ds(row0 + 5 * FLUSH, FLUSH)])
            pltpu.sync_copy(rows_v.at[pl.ds(0, ROW_CHUNK - 6 * FLUSH)],
                            acc.at[pl.ds(row0 + 6 * FLUSH,
                                         ROW_CHUNK - 6 * FLUSH)])

        @pl.when(s == NSUB - 1)
        def _():
            pltpu.sync_copy(rows_v.at[pl.ds(0, ROW_LAST - 5 * FLUSH)],
                            acc.at[pl.ds(row0 + 5 * FLUSH,
                                         ROW_LAST - 5 * FLUSH)])

        @pl.when(s < NSUB - 1)
        def _():
            pltpu.sync_copy(zbuf, cnt_sh.at[pl.ds(s * CNT_CHUNK, CNT_CHUNK)])

        @pl.when(s == NSUB - 1)
        def _():
            pltpu.sync_copy(zbuf.at[pl.ds(0, CNT_LAST)],
                            cnt_sh.at[pl.ds(s * CNT_CHUNK, CNT_LAST)])

        for k in range(CAP // LANES):
            src_buf[pl.ds(k * LANES, LANES)] = z16i
            dst_buf[pl.ds(k * LANES, LANES)] = z16i
        off_ref[0] = 0
        plsc.subcore_barrier()

        def _flush():
            for k in range(FLUSH // LANES):
                sl = pl.ds(k * LANES, LANES)
                src_dma[sl] = src_buf[sl]
                dst_dma[sl] = dst_buf[sl]
            pltpu.async_copy(x_hbm.at[src_dma], rows_v, sem).wait()
            pltpu.sync_copy(rows_v, acc.at[dst_dma], add=True)
            pltpu.sync_copy(ones_v, cnt_sh.at[dst_dma], add=True)

        @pl.loop(0, NBLK)
        def _(b):
            e0 = (b * NSUB + s) * BLKE
            pltpu.sync_copy(src_hbm.at[pl.ds(e0, BLKE)], blkb_src)
            pltpu.sync_copy(dst_hbm.at[pl.ds(e0, BLKE)], blkb_dst)

            @pl.loop(0, BLKE // LANES)
            def _(k):
                sl = pl.ds(k * LANES, LANES)
                d16 = blkb_dst[sl]
                s16 = blkb_src[sl]
                m = (d16 >= base) & (d16 < base + REG)
                mi = m.astype(jnp.int32)
                off0 = off_ref[0]
                pos = jnp.cumsum(mi) - mi + off0
                plsc.store_scatter(src_buf, [pos], s16, mask=m)
                plsc.store_scatter(dst_buf, [pos], d16 - base, mask=m)
                off1 = off0 + jnp.sum(mi)
                off_ref[0] = off1

                @pl.when(off1 >= FLUSH)
                def _():
                    _flush()
                    src_buf[pl.ds(0, LANES)] = src_buf[pl.ds(FLUSH, LANES)]
                    dst_buf[pl.ds(0, LANES)] = dst_buf[pl.ds(FLUSH, LANES)]
                    off_ref[0] = off1 - FLUSH

        offf = off_ref[0]

        @pl.when(offf > 0)
        def _():
            for k in range(FLUSH // LANES):
                sl = pl.ds(k * LANES, LANES)
                posv = lax.iota(jnp.int32, LANES) + (k * LANES)
                keep = posv < offf
                src_dma[sl] = src_buf[sl]
                dst_dma[sl] = jnp.where(
                    keep, dst_buf[sl], jnp.full((LANES,), DUMMY, jnp.int32))
            pltpu.async_copy(x_hbm.at[src_dma], rows_v, sem).wait()
            pltpu.sync_copy(rows_v, acc.at[dst_dma], add=True)
            pltpu.sync_copy(ones_v, cnt_sh.at[dst_dma], add=True)

        plsc.subcore_barrier()

        @pl.when(s < NSUB - 1)
        def _():
            pltpu.sync_copy(acc.at[pl.ds(row0, ROW_CHUNK)],
                            sum_hbm.at[pl.ds(base + row0, ROW_CHUNK)])

        @pl.when(s == NSUB - 1)
        def _():
            pltpu.sync_copy(acc.at[pl.ds(row0, ROW_LAST)],
                            sum_hbm.at[pl.ds(base + row0, ROW_LAST)])

        @pl.when(s < NSUB - 1)
        def _():
            pltpu.sync_copy(cnt_sh.at[pl.ds(s * CNT_CHUNK, CNT_CHUNK)], cbuf)
            pltpu.sync_copy(cbuf,
                            cnt_hbm.at[pl.ds(base + s * CNT_CHUNK, CNT_CHUNK)])

        @pl.when(s == NSUB - 1)
        def _():
            pltpu.sync_copy(cnt_sh.at[pl.ds(s * CNT_CHUNK, CNT_LAST)],
                            cbuf.at[pl.ds(0, CNT_LAST)])
            pltpu.sync_copy(cbuf.at[pl.ds(0, CNT_LAST)],
                            cnt_hbm.at[pl.ds(base + s * CNT_CHUNK, CNT_LAST)])

        plsc.subcore_barrier()


def _sc_segsum(x, src, dst):
    mesh = plsc.VectorSubcoreMesh(core_axis_name="c", subcore_axis_name="s")
    f = pl.kernel(
        _segsum_body,
        out_type=(jax.ShapeDtypeStruct((NPAD, D), jnp.float32),
                  jax.ShapeDtypeStruct((NPAD,), jnp.float32)),
        mesh=mesh,
        compiler_params=dataclasses.replace(
            pltpu.CompilerParams(), needs_layout_passes=False),
        scratch_types=[
            pltpu.VMEM((FLUSH, D), jnp.float32),
            pltpu.VMEM((BLKE,), jnp.int32),
            pltpu.VMEM((BLKE,), jnp.int32),
            pltpu.VMEM((CAP,), jnp.int32),
            pltpu.VMEM((CAP,), jnp.int32),
            pltpu.VMEM((FLUSH,), jnp.int32),
            pltpu.VMEM((FLUSH,), jnp.int32),
            pltpu.VMEM((CNT_CHUNK,), jnp.float32),
            pltpu.VMEM((FLUSH,), jnp.float32),
            pltpu.VMEM((CNT_CHUNK,), jnp.float32),
            pltpu.SMEM((1,), jnp.int32),
            pltpu.VMEM_SHARED((ACC_ROWS, D), jnp.float32),
            pltpu.VMEM_SHARED((ACC_ROWS,), jnp.float32),
            pltpu.SemaphoreType.DMA,
        ],
    )
    return f(x, src, dst)


def _dense_body(sum_ref, cnt_ref, x_ref, wl_ref, wr_ref, b_ref, o_ref):
    rcp = 1.0 / jnp.maximum(cnt_ref[...], 1.0)
    agg = sum_ref[...] * rcp
    y = lax.dot_general(agg, wl_ref[...], (((1,), (0,)), ((), ())),
                        precision=lax.Precision.HIGHEST,
                        preferred_element_type=jnp.float32)
    y = y + lax.dot_general(x_ref[...], wr_ref[...], (((1,), (0,)), ((), ())),
                            precision=lax.Precision.HIGHEST,
                            preferred_element_type=jnp.float32)
    o_ref[...] = jnp.maximum(y + b_ref[...], 0.0)


_BLKR = 1000


def _tc_dense(sum_pad, cnt_pad, x_dst, Wl, bl, Wr, gamma, beta):
    scale = gamma * _BN_SCALE
    wl = Wl.T * scale[None, :]
    wr = Wr.T * scale[None, :]
    bb = (bl * scale + beta).reshape(1, D)
    cnt2 = cnt_pad.reshape(-1, 1)
    return pl.pallas_call(
        _dense_body,
        grid=(N // _BLKR,),
        in_specs=[
            pl.BlockSpec((_BLKR, D), lambda i: (i, 0)),
            pl.BlockSpec((_BLKR, 1), lambda i: (i, 0)),
            pl.BlockSpec((_BLKR, D), lambda i: (i, 0)),
            pl.BlockSpec((D, D), lambda i: (0, 0)),
            pl.BlockSpec((D, D), lambda i: (0, 0)),
            pl.BlockSpec((1, D), lambda i: (0, 0)),
        ],
        out_specs=pl.BlockSpec((_BLKR, D), lambda i: (i, 0)),
        out_shape=jax.ShapeDtypeStruct((N, D), jnp.float32),
    )(sum_pad, cnt2, x_dst, wl, wr, bb)


def _pad_edges(edge):
    src = jnp.concatenate(
        [edge[0].astype(jnp.int32), jnp.zeros((EPAD - E_RAW,), jnp.int32)])
    dst = jnp.concatenate(
        [edge[1].astype(jnp.int32),
         jnp.full((EPAD - E_RAW,), SENTINEL, jnp.int32)])
    return src, dst


def kernel(x_user, x_event, edge_e2u, edge_u2e,
           Wl_u0, bl_u0, Wr_u0, gamma_u0, beta_u0,
           Wl_e0, bl_e0, Wr_e0, gamma_e0, beta_e0,
           Wl_u1, bl_u1, Wr_u1, gamma_u1, beta_u1,
           Wl_e1, bl_e1, Wr_e1, gamma_e1, beta_e1):
    se2u, de2u = _pad_edges(edge_e2u)
    su2e, du2e = _pad_edges(edge_u2e)
    params = {
        "u": [(Wl_u0, bl_u0, Wr_u0, gamma_u0, beta_u0),
              (Wl_u1, bl_u1, Wr_u1, gamma_u1, beta_u1)],
        "e": [(Wl_e0, bl_e0, Wr_e0, gamma_e0, beta_e0),
              (Wl_e1, bl_e1, Wr_e1, gamma_e1, beta_e1)],
    }
    xu, xe = x_user, x_event
    for i in range(2):
        su, cu = _sc_segsum(xe, se2u, de2u)
        xu = _tc_dense(su, cu, xu, *params["u"][i])
        se, ce = _sc_segsum(xu, su2e, du2e)
        xe = _tc_dense(se, ce, xe, *params["e"][i])
    return (xu, xe)


# speedup vs baseline: 8.5280x; 1.5680x over previous
import dataclasses
import functools
import math

import jax
import jax.numpy as jnp
from jax import lax
from jax.experimental import pallas as pl
from jax.experimental.pallas import tpu as pltpu
from jax.experimental.pallas import tpu_sc as plsc

D = 128
N = 50000
E_RAW = 600000
LANES = 16
NSUB = 16

BLKE = 1184
NBLK = 32
EPAD = BLKE * NSUB * NBLK
FLUSH = 128
ZROWS = 128
CAP = 160
REG = 12512
NPAD = 4 * REG
ACC_ROWS = 12528
DUMMY = REG
SENTINEL = 1 << 30
ROW_CHUNK = 784
ROW_LAST = REG - 15 * ROW_CHUNK
CNT_CHUNK = 784
CNT_LAST = REG - 15 * CNT_CHUNK

_BN_SCALE = 1.0 / math.sqrt(1.0 + 1e-5)


def _segsum_body(x_hbm, src_hbm, dst_hbm, sum_hbm, cnt_hbm,
                 rows_v, bs0, bd0, bs1, bd1, src_buf, dst_buf,
                 sdma0, ddma0, sdma1, ddma1, zbuf, ones_v, cbuf, off_ref,
                 acc, cnt_sh, semg, sembs0, sembd0, sembs1, sembd1):
    c = lax.axis_index("c")
    s = lax.axis_index("s")
    z16f = jnp.zeros((LANES,), jnp.float32)
    z16i = jnp.zeros((LANES,), jnp.int32)
    one16 = jnp.ones((LANES,), jnp.float32)
    row0 = s * ROW_CHUNK

    @pl.loop(0, CNT_CHUNK // LANES)
    def _(i):
        zbuf[pl.ds(i * LANES, LANES)] = z16f

    for k in range(FLUSH // LANES):
        ones_v[pl.ds(k * LANES, LANES)] = one16

    for p in range(2):
        base = (2 * p + c) * REG

        @pl.loop(0, ZROWS)
        def _(r):
            for k in range(D // LANES):
                rows_v[r, pl.ds(k * LANES, LANES)] = z16f

        zstage = rows_v.at[pl.ds(0, ZROWS)]
        for k in range(5):
            pltpu.sync_copy(zstage, acc.at[pl.ds(row0 + k * ZROWS, ZROWS)])

        @pl.when(s < NSUB - 1)
        def _():
            pltpu.sync_copy(zstage, acc.at[pl.ds(row0 + 5 * ZROWS, ZROWS)])
            pltpu.sync_copy(rows_v.at[pl.ds(0, ROW_CHUNK - 6 * ZROWS)],
                            acc.at[pl.ds(row0 + 6 * ZROWS,
                                         ROW_CHUNK - 6 * ZROWS)])

        @pl.when(s == NSUB - 1)
        def _():
            pltpu.sync_copy(rows_v.at[pl.ds(0, ROW_LAST - 5 * ZROWS)],
                            acc.at[pl.ds(row0 + 5 * ZROWS,
                                         ROW_LAST - 5 * ZROWS)])

        @pl.when(s < NSUB - 1)
        def _():
            pltpu.sync_copy(zbuf, cnt_sh.at[pl.ds(s * CNT_CHUNK, CNT_CHUNK)])

        @pl.when(s == NSUB - 1)
        def _():
            pltpu.sync_copy(zbuf.at[pl.ds(0, CNT_LAST)],
                            cnt_sh.at[pl.ds(s * CNT_CHUNK, CNT_LAST)])

        for k in range(CAP // LANES):
            src_buf[pl.ds(k * LANES, LANES)] = z16i
            dst_buf[pl.ds(k * LANES, LANES)] = z16i
        off_ref[0] = 0
        off_ref[1] = 0
        plsc.subcore_barrier()

        def _blk_start(b, bs, bd, ss, sd):
            e0 = (b * NSUB + s) * BLKE
            pltpu.async_copy(src_hbm.at[pl.ds(e0, BLKE)], bs, ss)
            pltpu.async_copy(dst_hbm.at[pl.ds(e0, BLKE)], bd, sd)

        def _blk_wait(b, bs, bd, ss, sd):
            e0 = (b * NSUB + s) * BLKE
            pltpu.make_async_copy(src_hbm.at[pl.ds(e0, BLKE)], bs, ss).wait()
            pltpu.make_async_copy(dst_hbm.at[pl.ds(e0, BLKE)], bd, sd).wait()

        def _do_scatter(sd, dd):
            pltpu.make_async_copy(x_hbm.at[sd], rows_v, semg).wait()
            pltpu.sync_copy(rows_v, acc.at[dd], add=True)
            pltpu.sync_copy(ones_v, cnt_sh.at[dd], add=True)

        def _flush(q, fc):
            sd, dd = (sdma0, ddma0) if q == 0 else (sdma1, ddma1)
            od, odd = (sdma1, ddma1) if q == 0 else (sdma0, ddma0)
            for k in range(FLUSH // LANES):
                sl = pl.ds(k * LANES, LANES)
                sd[sl] = src_buf[sl]
                dd[sl] = dst_buf[sl]

            @pl.when(fc > 0)
            def _():
                _do_scatter(od, odd)
            pltpu.async_copy(x_hbm.at[sd], rows_v, semg)

        def _chunks(bs, bd):
            @pl.loop(0, BLKE // LANES)
            def _(k):
                sl = pl.ds(k * LANES, LANES)
                d16 = bd[sl]
                s16 = bs[sl]
                m = (d16 >= base) & (d16 < base + REG)
                mi = m.astype(jnp.int32)
                off0 = off_ref[0]
                pos = jnp.cumsum(mi) - mi + off0
                plsc.store_scatter(src_buf, [pos], s16, mask=m)
                plsc.store_scatter(dst_buf, [pos], d16 - base, mask=m)
                off1 = off0 + jnp.sum(mi)
                off_ref[0] = off1

                @pl.when(off1 >= FLUSH)
                def _():
                    fc = off_ref[1]

                    @pl.when(fc % 2 == 0)
                    def _():
                        _flush(0, fc)

                    @pl.when(fc % 2 == 1)
                    def _():
                        _flush(1, fc)
                    src_buf[pl.ds(0, LANES)] = src_buf[pl.ds(FLUSH, LANES)]
                    dst_buf[pl.ds(0, LANES)] = dst_buf[pl.ds(FLUSH, LANES)]
                    off_ref[0] = off1 - FLUSH
                    off_ref[1] = fc + 1

        _blk_start(0, bs0, bd0, sembs0, sembd0)

        @pl.loop(0, NBLK // 2)
        def _(t):
            b0 = 2 * t
            _blk_wait(b0, bs0, bd0, sembs0, sembd0)
            _blk_start(b0 + 1, bs1, bd1, sembs1, sembd1)
            _chunks(bs0, bd0)
            _blk_wait(b0 + 1, bs1, bd1, sembs1, sembd1)

            @pl.when(t < NBLK // 2 - 1)
            def _():
                _blk_start(b0 + 2, bs0, bd0, sembs0, sembd0)
            _chunks(bs1, bd1)

        fcf = off_ref[1]

        @pl.when((fcf > 0) & (fcf % 2 == 1))
        def _():
            _do_scatter(sdma0, ddma0)

        @pl.when((fcf > 0) & (fcf % 2 == 0))
        def _():
            _do_scatter(sdma1, ddma1)

        offf = off_ref[0]

        @pl.when(offf > 0)
        def _():
            for k in range(FLUSH // LANES):
                sl = pl.ds(k * LANES, LANES)
                posv = lax.iota(jnp.int32, LANES) + (k * LANES)
                keep = posv < offf
                sdma0[sl] = src_buf[sl]
                ddma0[sl] = jnp.where(
                    keep, dst_buf[sl], jnp.full((LANES,), DUMMY, jnp.int32))
            pltpu.async_copy(x_hbm.at[sdma0], rows_v, semg).wait()
            pltpu.sync_copy(rows_v, acc.at[ddma0], add=True)
            pltpu.sync_copy(ones_v, cnt_sh.at[ddma0], add=True)

        plsc.subcore_barrier()

        @pl.when(s < NSUB - 1)
        def _():
            pltpu.sync_copy(acc.at[pl.ds(row0, ROW_CHUNK)],
                            sum_hbm.at[pl.ds(base + row0, ROW_CHUNK)])

        @pl.when(s == NSUB - 1)
        def _():
            pltpu.sync_copy(acc.at[pl.ds(row0, ROW_LAST)],
                            sum_hbm.at[pl.ds(base + row0, ROW_LAST)])

        @pl.when(s < NSUB - 1)
        def _():
            pltpu.sync_copy(cnt_sh.at[pl.ds(s * CNT_CHUNK, CNT_CHUNK)], cbuf)
            pltpu.sync_copy(cbuf,
                            cnt_hbm.at[pl.ds(base + s * CNT_CHUNK, CNT_CHUNK)])

        @pl.when(s == NSUB - 1)
        def _():
            pltpu.sync_copy(cnt_sh.at[pl.ds(s * CNT_CHUNK, CNT_LAST)],
                            cbuf.at[pl.ds(0, CNT_LAST)])
            pltpu.sync_copy(cbuf.at[pl.ds(0, CNT_LAST)],
                            cnt_hbm.at[pl.ds(base + s * CNT_CHUNK, CNT_LAST)])

        plsc.subcore_barrier()


def _sc_segsum(x, src, dst):
    mesh = plsc.VectorSubcoreMesh(core_axis_name="c", subcore_axis_name="s")
    f = pl.kernel(
        _segsum_body,
        out_type=(jax.ShapeDtypeStruct((NPAD, D), jnp.float32),
                  jax.ShapeDtypeStruct((NPAD,), jnp.float32)),
        mesh=mesh,
        compiler_params=dataclasses.replace(
            pltpu.CompilerParams(), needs_layout_passes=False),
        scratch_types=[
            pltpu.VMEM((FLUSH, D), jnp.float32),
            pltpu.VMEM((BLKE,), jnp.int32),
            pltpu.VMEM((BLKE,), jnp.int32),
            pltpu.VMEM((BLKE,), jnp.int32),
            pltpu.VMEM((BLKE,), jnp.int32),
            pltpu.VMEM((CAP,), jnp.int32),
            pltpu.VMEM((CAP,), jnp.int32),
            pltpu.VMEM((FLUSH,), jnp.int32),
            pltpu.VMEM((FLUSH,), jnp.int32),
            pltpu.VMEM((FLUSH,), jnp.int32),
            pltpu.VMEM((FLUSH,), jnp.int32),
            pltpu.VMEM((CNT_CHUNK,), jnp.float32),
            pltpu.VMEM((FLUSH,), jnp.float32),
            pltpu.VMEM((CNT_CHUNK,), jnp.float32),
            pltpu.SMEM((2,), jnp.int32),
            pltpu.VMEM_SHARED((ACC_ROWS, D), jnp.float32),
            pltpu.VMEM_SHARED((ACC_ROWS,), jnp.float32),
            pltpu.SemaphoreType.DMA,
            pltpu.SemaphoreType.DMA,
            pltpu.SemaphoreType.DMA,
            pltpu.SemaphoreType.DMA,
            pltpu.SemaphoreType.DMA,
        ],
    )
    return f(x, src, dst)


def _dense_body(sum_ref, cnt_ref, x_ref, wl_ref, wr_ref, b_ref, o_ref):
    rcp = 1.0 / jnp.maximum(cnt_ref[...], 1.0)
    agg = sum_ref[...] * rcp
    y = lax.dot_general(agg, wl_ref[...], (((1,), (0,)), ((), ())),
                        precision=lax.Precision.HIGHEST,
                        preferred_element_type=jnp.float32)
    y = y + lax.dot_general(x_ref[...], wr_ref[...], (((1,), (0,)), ((), ())),
                            precision=lax.Precision.HIGHEST,
                            preferred_element_type=jnp.float32)
    o_ref[...] = jnp.maximum(y + b_ref[...], 0.0)


_BLKR = 1000


def _tc_dense(sum_pad, cnt_pad, x_dst, Wl, bl, Wr, gamma, beta):
    scale = gamma * _BN_SCALE
    wl = Wl.T * scale[None, :]
    wr = Wr.T * scale[None, :]
    bb = (bl * scale + beta).reshape(1, D)
    cnt2 = cnt_pad.reshape(-1, 1)
    return pl.pallas_call(
        _dense_body,
        grid=(N // _BLKR,),
        in_specs=[
            pl.BlockSpec((_BLKR, D), lambda i: (i, 0)),
            pl.BlockSpec((_BLKR, 1), lambda i: (i, 0)),
            pl.BlockSpec((_BLKR, D), lambda i: (i, 0)),
            pl.BlockSpec((D, D), lambda i: (0, 0)),
            pl.BlockSpec((D, D), lambda i: (0, 0)),
            pl.BlockSpec((1, D), lambda i: (0, 0)),
        ],
        out_specs=pl.BlockSpec((_BLKR, D), lambda i: (i, 0)),
        out_shape=jax.ShapeDtypeStruct((N, D), jnp.float32),
    )(sum_pad, cnt2, x_dst, wl, wr, bb)


def _pad_edges(edge):
    src = jnp.concatenate(
        [edge[0].astype(jnp.int32), jnp.zeros((EPAD - E_RAW,), jnp.int32)])
    dst = jnp.concatenate(
        [edge[1].astype(jnp.int32),
         jnp.full((EPAD - E_RAW,), SENTINEL, jnp.int32)])
    return src, dst


def kernel(x_user, x_event, edge_e2u, edge_u2e,
           Wl_u0, bl_u0, Wr_u0, gamma_u0, beta_u0,
           Wl_e0, bl_e0, Wr_e0, gamma_e0, beta_e0,
           Wl_u1, bl_u1, Wr_u1, gamma_u1, beta_u1,
           Wl_e1, bl_e1, Wr_e1, gamma_e1, beta_e1):
    se2u, de2u = _pad_edges(edge_e2u)
    su2e, du2e = _pad_edges(edge_u2e)
    params = {
        "u": [(Wl_u0, bl_u0, Wr_u0, gamma_u0, beta_u0),
              (Wl_u1, bl_u1, Wr_u1, gamma_u1, beta_u1)],
        "e": [(Wl_e0, bl_e0, Wr_e0, gamma_e0, beta_e0),
              (Wl_e1, bl_e1, Wr_e1, gamma_e1, beta_e1)],
    }
    xu, xe = x_user, x_event
    for i in range(2):
        su, cu = _sc_segsum(xe, se2u, de2u)
        xu = _tc_dense(su, cu, xu, *params["u"][i])
        se, ce = _sc_segsum(xu, su2e, du2e)
        xe = _tc_dense(se, ce, xe, *params["e"][i])
    return (xu, xe)


# speedup vs baseline: 9.2008x; 1.0789x over previous
import dataclasses
import functools
import math

import jax
import jax.numpy as jnp
from jax import lax
from jax.experimental import pallas as pl
from jax.experimental.pallas import tpu as pltpu
from jax.experimental.pallas import tpu_sc as plsc

D = 128
N = 50000
E_RAW = 600000
LANES = 16
NSUB = 16

BLKE = 1184
NBLK = 32
EPAD = BLKE * NSUB * NBLK
FLUSH = 128
ZROWS = 128
CAP = 160
REG = 12512
NPAD = 4 * REG
ACC_ROWS = 12528
DUMMY = REG
SENTINEL = 1 << 30
ROW_CHUNK = 784
ROW_LAST = REG - 15 * ROW_CHUNK
CNT_CHUNK = 784
CNT_LAST = REG - 15 * CNT_CHUNK

_BN_SCALE = 1.0 / math.sqrt(1.0 + 1e-5)


def _segsum_body(x_hbm, src_hbm, dst_hbm, sum_hbm, cnt_hbm,
                 rows_v, bs0, bd0, bs1, bd1, src_buf, dst_buf,
                 sdma0, ddma0, sdma1, ddma1, zbuf, ones_v, cbuf, off_ref,
                 acc, cnt_sh, semg, sembs0, sembd0, sembs1, sembd1,
                 semc0, semc1):
    c = lax.axis_index("c")
    s = lax.axis_index("s")
    z16f = jnp.zeros((LANES,), jnp.float32)
    z16i = jnp.zeros((LANES,), jnp.int32)
    one16 = jnp.ones((LANES,), jnp.float32)
    row0 = s * ROW_CHUNK

    @pl.loop(0, CNT_CHUNK // LANES)
    def _(i):
        zbuf[pl.ds(i * LANES, LANES)] = z16f

    for k in range(FLUSH // LANES):
        ones_v[pl.ds(k * LANES, LANES)] = one16

    for p in range(2):
        base = (2 * p + c) * REG

        @pl.loop(0, ZROWS)
        def _(r):
            for k in range(D // LANES):
                rows_v[r, pl.ds(k * LANES, LANES)] = z16f

        zstage = rows_v.at[pl.ds(0, ZROWS)]
        for k in range(5):
            pltpu.sync_copy(zstage, acc.at[pl.ds(row0 + k * ZROWS, ZROWS)])

        @pl.when(s < NSUB - 1)
        def _():
            pltpu.sync_copy(zstage, acc.at[pl.ds(row0 + 5 * ZROWS, ZROWS)])
            pltpu.sync_copy(rows_v.at[pl.ds(0, ROW_CHUNK - 6 * ZROWS)],
                            acc.at[pl.ds(row0 + 6 * ZROWS,
                                         ROW_CHUNK - 6 * ZROWS)])

        @pl.when(s == NSUB - 1)
        def _():
            pltpu.sync_copy(rows_v.at[pl.ds(0, ROW_LAST - 5 * ZROWS)],
                            acc.at[pl.ds(row0 + 5 * ZROWS,
                                         ROW_LAST - 5 * ZROWS)])

        @pl.when(s < NSUB - 1)
        def _():
            pltpu.sync_copy(zbuf, cnt_sh.at[pl.ds(s * CNT_CHUNK, CNT_CHUNK)])

        @pl.when(s == NSUB - 1)
        def _():
            pltpu.sync_copy(zbuf.at[pl.ds(0, CNT_LAST)],
                            cnt_sh.at[pl.ds(s * CNT_CHUNK, CNT_LAST)])

        for k in range(CAP // LANES):
            src_buf[pl.ds(k * LANES, LANES)] = z16i
            dst_buf[pl.ds(k * LANES, LANES)] = z16i
        off_ref[0] = 0
        off_ref[1] = 0
        plsc.subcore_barrier()

        def _blk_start(b, bs, bd, ss, sd):
            e0 = (b * NSUB + s) * BLKE
            pltpu.async_copy(src_hbm.at[pl.ds(e0, BLKE)], bs, ss)
            pltpu.async_copy(dst_hbm.at[pl.ds(e0, BLKE)], bd, sd)

        def _blk_wait(b, bs, bd, ss, sd):
            e0 = (b * NSUB + s) * BLKE
            pltpu.make_async_copy(src_hbm.at[pl.ds(e0, BLKE)], bs, ss).wait()
            pltpu.make_async_copy(dst_hbm.at[pl.ds(e0, BLKE)], bd, sd).wait()

        def _do_scatter(sd, dd, cnt_sem):
            pltpu.make_async_copy(x_hbm.at[sd], rows_v, semg).wait()
            pltpu.sync_copy(rows_v, acc.at[dd], add=True)
            if cnt_sem is None:
                pltpu.sync_copy(ones_v, cnt_sh.at[dd], add=True)
            else:
                pltpu.async_copy(ones_v, cnt_sh.at[dd], cnt_sem, add=True)

        def _flush(q, fc):
            sd, dd = (sdma0, ddma0) if q == 0 else (sdma1, ddma1)
            od, odd = (sdma1, ddma1) if q == 0 else (sdma0, ddma0)
            semc, osemc = (semc0, semc1) if q == 0 else (semc1, semc0)

            @pl.when(fc >= 2)
            def _():
                pltpu.make_async_copy(ones_v, cnt_sh.at[dd], semc).wait()
            for k in range(FLUSH // LANES):
                sl = pl.ds(k * LANES, LANES)
                sd[sl] = src_buf[sl]
                dd[sl] = dst_buf[sl]

            @pl.when(fc > 0)
            def _():
                _do_scatter(od, odd, osemc)
            pltpu.async_copy(x_hbm.at[sd], rows_v, semg)

        def _chunks(bs, bd):
            @pl.loop(0, BLKE // LANES)
            def _(k):
                sl = pl.ds(k * LANES, LANES)
                d16 = bd[sl]
                s16 = bs[sl]
                m = (d16 >= base) & (d16 < base + REG)
                mi = m.astype(jnp.int32)
                off0 = off_ref[0]
                pos = jnp.cumsum(mi) - mi + off0
                plsc.store_scatter(src_buf, [pos], s16, mask=m)
                plsc.store_scatter(dst_buf, [pos], d16 - base, mask=m)
                off1 = off0 + jnp.sum(mi)
                off_ref[0] = off1

                @pl.when(off1 >= FLUSH)
                def _():
                    fc = off_ref[1]

                    @pl.when(fc % 2 == 0)
                    def _():
                        _flush(0, fc)

                    @pl.when(fc % 2 == 1)
                    def _():
                        _flush(1, fc)
                    src_buf[pl.ds(0, LANES)] = src_buf[pl.ds(FLUSH, LANES)]
                    dst_buf[pl.ds(0, LANES)] = dst_buf[pl.ds(FLUSH, LANES)]
                    off_ref[0] = off1 - FLUSH
                    off_ref[1] = fc + 1

        _blk_start(0, bs0, bd0, sembs0, sembd0)

        @pl.loop(0, NBLK // 2)
        def _(t):
            b0 = 2 * t
            _blk_wait(b0, bs0, bd0, sembs0, sembd0)
            _blk_start(b0 + 1, bs1, bd1, sembs1, sembd1)
            _chunks(bs0, bd0)
            _blk_wait(b0 + 1, bs1, bd1, sembs1, sembd1)

            @pl.when(t < NBLK // 2 - 1)
            def _():
                _blk_start(b0 + 2, bs0, bd0, sembs0, sembd0)
            _chunks(bs1, bd1)

        fcf = off_ref[1]

        @pl.when((fcf > 0) & (fcf % 2 == 1))
        def _():
            _do_scatter(sdma0, ddma0, None)

        @pl.when((fcf > 0) & (fcf % 2 == 0))
        def _():
            _do_scatter(sdma1, ddma1, None)

        @pl.when((fcf >= 2) & (fcf % 2 == 0))
        def _():
            pltpu.make_async_copy(ones_v, cnt_sh.at[ddma0], semc0).wait()

        @pl.when((fcf >= 2) & (fcf % 2 == 1))
        def _():
            pltpu.make_async_copy(ones_v, cnt_sh.at[ddma1], semc1).wait()

        offf = off_ref[0]

        @pl.when(offf > 0)
        def _():
            for k in range(FLUSH // LANES):
                sl = pl.ds(k * LANES, LANES)
                posv = lax.iota(jnp.int32, LANES) + (k * LANES)
                keep = posv < offf
                sdma0[sl] = src_buf[sl]
                ddma0[sl] = jnp.where(
                    keep, dst_buf[sl], jnp.full((LANES,), DUMMY, jnp.int32))
            pltpu.async_copy(x_hbm.at[sdma0], rows_v, semg).wait()
            pltpu.sync_copy(rows_v, acc.at[ddma0], add=True)
            pltpu.sync_copy(ones_v, cnt_sh.at[ddma0], add=True)

        plsc.subcore_barrier()

        @pl.when(s < NSUB - 1)
        def _():
            pltpu.sync_copy(acc.at[pl.ds(row0, ROW_CHUNK)],
                            sum_hbm.at[pl.ds(base + row0, ROW_CHUNK)])

        @pl.when(s == NSUB - 1)
        def _():
            pltpu.sync_copy(acc.at[pl.ds(row0, ROW_LAST)],
                            sum_hbm.at[pl.ds(base + row0, ROW_LAST)])

        @pl.when(s < NSUB - 1)
        def _():
            pltpu.sync_copy(cnt_sh.at[pl.ds(s * CNT_CHUNK, CNT_CHUNK)], cbuf)
            pltpu.sync_copy(cbuf,
                            cnt_hbm.at[pl.ds(base + s * CNT_CHUNK, CNT_CHUNK)])

        @pl.when(s == NSUB - 1)
        def _():
            pltpu.sync_copy(cnt_sh.at[pl.ds(s * CNT_CHUNK, CNT_LAST)],
                            cbuf.at[pl.ds(0, CNT_LAST)])
            pltpu.sync_copy(cbuf.at[pl.ds(0, CNT_LAST)],
                            cnt_hbm.at[pl.ds(base + s * CNT_CHUNK, CNT_LAST)])

        plsc.subcore_barrier()


def _sc_segsum(x, src, dst):
    mesh = plsc.VectorSubcoreMesh(core_axis_name="c", subcore_axis_name="s")
    f = pl.kernel(
        _segsum_body,
        out_type=(jax.ShapeDtypeStruct((NPAD, D), jnp.float32),
                  jax.ShapeDtypeStruct((NPAD,), jnp.float32)),
        mesh=mesh,
        compiler_params=dataclasses.replace(
            pltpu.CompilerParams(), needs_layout_passes=False),
        scratch_types=[
            pltpu.VMEM((FLUSH, D), jnp.float32),
            pltpu.VMEM((BLKE,), jnp.int32),
            pltpu.VMEM((BLKE,), jnp.int32),
            pltpu.VMEM((BLKE,), jnp.int32),
            pltpu.VMEM((BLKE,), jnp.int32),
            pltpu.VMEM((CAP,), jnp.int32),
            pltpu.VMEM((CAP,), jnp.int32),
            pltpu.VMEM((FLUSH,), jnp.int32),
            pltpu.VMEM((FLUSH,), jnp.int32),
            pltpu.VMEM((FLUSH,), jnp.int32),
            pltpu.VMEM((FLUSH,), jnp.int32),
            pltpu.VMEM((CNT_CHUNK,), jnp.float32),
            pltpu.VMEM((FLUSH,), jnp.float32),
            pltpu.VMEM((CNT_CHUNK,), jnp.float32),
            pltpu.SMEM((2,), jnp.int32),
            pltpu.VMEM_SHARED((ACC_ROWS, D), jnp.float32),
            pltpu.VMEM_SHARED((ACC_ROWS,), jnp.float32),
            pltpu.SemaphoreType.DMA,
            pltpu.SemaphoreType.DMA,
            pltpu.SemaphoreType.DMA,
            pltpu.SemaphoreType.DMA,
            pltpu.SemaphoreType.DMA,
            pltpu.SemaphoreType.DMA,
            pltpu.SemaphoreType.DMA,
        ],
    )
    return f(x, src, dst)


_BLKR = 1000


def _linr_body(x_ref, wr_ref, o_ref):
    o_ref[...] = lax.dot_general(x_ref[...], wr_ref[...],
                                 (((1,), (0,)), ((), ())),
                                 precision=lax.Precision.HIGHEST,
                                 preferred_element_type=jnp.float32)


def _tc_linr(x_dst, wr):
    return pl.pallas_call(
        _linr_body,
        grid=(N // _BLKR,),
        in_specs=[
            pl.BlockSpec((_BLKR, D), lambda i: (i, 0)),
            pl.BlockSpec((D, D), lambda i: (0, 0)),
        ],
        out_specs=pl.BlockSpec((_BLKR, D), lambda i: (i, 0)),
        out_shape=jax.ShapeDtypeStruct((N, D), jnp.float32),
    )(x_dst, wr)


def _finish_body(sum_ref, cnt_ref, lr_ref, wl_ref, b_ref, o_ref):
    rcp = 1.0 / jnp.maximum(cnt_ref[...], 1.0)
    agg = sum_ref[...] * rcp
    y = lax.dot_general(agg, wl_ref[...], (((1,), (0,)), ((), ())),
                        precision=lax.Precision.HIGHEST,
                        preferred_element_type=jnp.float32)
    o_ref[...] = jnp.maximum(y + lr_ref[...] + b_ref[...], 0.0)


def _tc_finish(sum_pad, cnt_pad, linr, wl, bb):
    cnt2 = cnt_pad.reshape(-1, 1)
    return pl.pallas_call(
        _finish_body,
        grid=(N // _BLKR,),
        in_specs=[
            pl.BlockSpec((_BLKR, D), lambda i: (i, 0)),
            pl.BlockSpec((_BLKR, 1), lambda i: (i, 0)),
            pl.BlockSpec((_BLKR, D), lambda i: (i, 0)),
            pl.BlockSpec((D, D), lambda i: (0, 0)),
            pl.BlockSpec((1, D), lambda i: (0, 0)),
        ],
        out_specs=pl.BlockSpec((_BLKR, D), lambda i: (i, 0)),
        out_shape=jax.ShapeDtypeStruct((N, D), jnp.float32),
    )(sum_pad, cnt2, linr, wl, bb)


def _pad_edges(edge):
    src = jnp.concatenate(
        [edge[0].astype(jnp.int32), jnp.zeros((EPAD - E_RAW,), jnp.int32)])
    dst = jnp.concatenate(
        [edge[1].astype(jnp.int32),
         jnp.full((EPAD - E_RAW,), SENTINEL, jnp.int32)])
    return src, dst


def kernel(x_user, x_event, edge_e2u, edge_u2e,
           Wl_u0, bl_u0, Wr_u0, gamma_u0, beta_u0,
           Wl_e0, bl_e0, Wr_e0, gamma_e0, beta_e0,
           Wl_u1, bl_u1, Wr_u1, gamma_u1, beta_u1,
           Wl_e1, bl_e1, Wr_e1, gamma_e1, beta_e1):
    se2u, de2u = _pad_edges(edge_e2u)
    su2e, du2e = _pad_edges(edge_u2e)
    params = {
        "u": [(Wl_u0, bl_u0, Wr_u0, gamma_u0, beta_u0),
              (Wl_u1, bl_u1, Wr_u1, gamma_u1, beta_u1)],
        "e": [(Wl_e0, bl_e0, Wr_e0, gamma_e0, beta_e0),
              (Wl_e1, bl_e1, Wr_e1, gamma_e1, beta_e1)],
    }
    def _prep(Wl, bl, Wr, gamma, beta):
        scale = gamma * _BN_SCALE
        return (Wl.T * scale[None, :], Wr.T * scale[None, :],
                (bl * scale + beta).reshape(1, D))

    xu, xe = x_user, x_event
    for i in range(2):
        wl, wr, bb = _prep(*params["u"][i])
        lr = _tc_linr(xu, wr)
        su, cu = _sc_segsum(xe, se2u, de2u)
        xu = _tc_finish(su, cu, lr, wl, bb)
        wl, wr, bb = _prep(*params["e"][i])
        lr = _tc_linr(xe, wr)
        se, ce = _sc_segsum(xu, su2e, du2e)
        xe = _tc_finish(se, ce, lr, wl, bb)
    return (xu, xe)


# speedup vs baseline: 9.8134x; 1.0666x over previous
import dataclasses
import functools
import math

import jax
import jax.numpy as jnp
from jax import lax
from jax.experimental import pallas as pl
from jax.experimental.pallas import tpu as pltpu
from jax.experimental.pallas import tpu_sc as plsc

D = 128
N = 50000
E_RAW = 600000
LANES = 16
NSUB = 16

BLKE = 1184
NBLK = 32
EPAD = BLKE * NSUB * NBLK
FLUSH = 128
ZROWS = 128
CAP = 160
REG = 12512
NPAD = 4 * REG
ACC_ROWS = 12528
DUMMY = REG
SENTINEL = 1 << 30
ROW_CHUNK = 784
ROW_LAST = REG - 15 * ROW_CHUNK
CNT_CHUNK = 784
CNT_LAST = REG - 15 * CNT_CHUNK

_BN_SCALE = 1.0 / math.sqrt(1.0 + 1e-5)


def _segsum_body(x_hbm, src_hbm, dst_hbm, sum_hbm, cnt_hbm,
                 rows_v, bs0, bd0, bs1, bd1, src_buf, dst_buf,
                 sdma0, ddma0, sdma1, ddma1, zbuf, ones_v, cbuf, off_ref,
                 acc, cnt_sh, semg, sembs0, sembd0, sembs1, sembd1,
                 semc0, semc1):
    c = lax.axis_index("c")
    s = lax.axis_index("s")
    z16f = jnp.zeros((LANES,), jnp.float32)
    z16i = jnp.zeros((LANES,), jnp.int32)
    one16 = jnp.ones((LANES,), jnp.float32)
    row0 = s * ROW_CHUNK

    @pl.loop(0, CNT_CHUNK // LANES)
    def _(i):
        zbuf[pl.ds(i * LANES, LANES)] = z16f

    for k in range(FLUSH // LANES):
        ones_v[pl.ds(k * LANES, LANES)] = one16

    for p in range(2):
        base = (2 * p + c) * REG

        @pl.loop(0, ZROWS)
        def _(r):
            for k in range(D // LANES):
                rows_v[r, pl.ds(k * LANES, LANES)] = z16f

        zstage = rows_v.at[pl.ds(0, ZROWS)]
        for k in range(5):
            pltpu.sync_copy(zstage, acc.at[pl.ds(row0 + k * ZROWS, ZROWS)])

        @pl.when(s < NSUB - 1)
        def _():
            pltpu.sync_copy(zstage, acc.at[pl.ds(row0 + 5 * ZROWS, ZROWS)])
            pltpu.sync_copy(rows_v.at[pl.ds(0, ROW_CHUNK - 6 * ZROWS)],
                            acc.at[pl.ds(row0 + 6 * ZROWS,
                                         ROW_CHUNK - 6 * ZROWS)])

        @pl.when(s == NSUB - 1)
        def _():
            pltpu.sync_copy(rows_v.at[pl.ds(0, ROW_LAST - 5 * ZROWS)],
                            acc.at[pl.ds(row0 + 5 * ZROWS,
                                         ROW_LAST - 5 * ZROWS)])

        @pl.when(s < NSUB - 1)
        def _():
            pltpu.sync_copy(zbuf, cnt_sh.at[pl.ds(s * CNT_CHUNK, CNT_CHUNK)])

        @pl.when(s == NSUB - 1)
        def _():
            pltpu.sync_copy(zbuf.at[pl.ds(0, CNT_LAST)],
                            cnt_sh.at[pl.ds(s * CNT_CHUNK, CNT_LAST)])

        for k in range(CAP // LANES):
            src_buf[pl.ds(k * LANES, LANES)] = z16i
            dst_buf[pl.ds(k * LANES, LANES)] = z16i
        off_ref[0] = 0
        off_ref[1] = 0
        plsc.subcore_barrier()

        def _blk_start(b, bs, bd, ss, sd):
            e0 = (b * NSUB + s) * BLKE
            pltpu.async_copy(src_hbm.at[pl.ds(e0, BLKE)], bs, ss)
            pltpu.async_copy(dst_hbm.at[pl.ds(e0, BLKE)], bd, sd)

        def _blk_wait(b, bs, bd, ss, sd):
            e0 = (b * NSUB + s) * BLKE
            pltpu.make_async_copy(src_hbm.at[pl.ds(e0, BLKE)], bs, ss).wait()
            pltpu.make_async_copy(dst_hbm.at[pl.ds(e0, BLKE)], bd, sd).wait()

        def _do_scatter(sd, dd, cnt_sem):
            pltpu.make_async_copy(x_hbm.at[sd], rows_v, semg).wait()
            pltpu.sync_copy(rows_v, acc.at[dd], add=True)
            if cnt_sem is None:
                pltpu.sync_copy(ones_v, cnt_sh.at[dd], add=True)
            else:
                pltpu.async_copy(ones_v, cnt_sh.at[dd], cnt_sem, add=True)

        def _flush(q, fc):
            sd, dd = (sdma0, ddma0) if q == 0 else (sdma1, ddma1)
            od, odd = (sdma1, ddma1) if q == 0 else (sdma0, ddma0)
            semc, osemc = (semc0, semc1) if q == 0 else (semc1, semc0)

            @pl.when(fc >= 2)
            def _():
                pltpu.make_async_copy(ones_v, cnt_sh.at[dd], semc).wait()
            for k in range(FLUSH // LANES):
                sl = pl.ds(k * LANES, LANES)
                sd[sl] = src_buf[sl]
                dd[sl] = dst_buf[sl]

            @pl.when(fc > 0)
            def _():
                _do_scatter(od, odd, osemc)
            pltpu.async_copy(x_hbm.at[sd], rows_v, semg)

        def _chunks(bs, bd):
            @pl.loop(0, BLKE // LANES)
            def _(k):
                sl = pl.ds(k * LANES, LANES)
                d16 = bd[sl]
                s16 = bs[sl]
                m = (d16 >= base) & (d16 < base + REG)
                mi = m.astype(jnp.int32)
                off0 = off_ref[0]
                pos = jnp.cumsum(mi) - mi + off0
                plsc.store_scatter(src_buf, [pos], s16, mask=m)
                plsc.store_scatter(dst_buf, [pos], d16 - base, mask=m)
                off1 = off0 + jnp.sum(mi)
                off_ref[0] = off1

                @pl.when(off1 >= FLUSH)
                def _():
                    fc = off_ref[1]

                    @pl.when(fc % 2 == 0)
                    def _():
                        _flush(0, fc)

                    @pl.when(fc % 2 == 1)
                    def _():
                        _flush(1, fc)
                    src_buf[pl.ds(0, LANES)] = src_buf[pl.ds(FLUSH, LANES)]
                    dst_buf[pl.ds(0, LANES)] = dst_buf[pl.ds(FLUSH, LANES)]
                    off_ref[0] = off1 - FLUSH
                    off_ref[1] = fc + 1

        _blk_start(0, bs0, bd0, sembs0, sembd0)

        @pl.loop(0, NBLK // 2)
        def _(t):
            b0 = 2 * t
            _blk_wait(b0, bs0, bd0, sembs0, sembd0)
            _blk_start(b0 + 1, bs1, bd1, sembs1, sembd1)
            _chunks(bs0, bd0)
            _blk_wait(b0 + 1, bs1, bd1, sembs1, sembd1)

            @pl.when(t < NBLK // 2 - 1)
            def _():
                _blk_start(b0 + 2, bs0, bd0, sembs0, sembd0)
            _chunks(bs1, bd1)

        fcf = off_ref[1]

        @pl.when((fcf > 0) & (fcf % 2 == 1))
        def _():
            _do_scatter(sdma0, ddma0, None)

        @pl.when((fcf > 0) & (fcf % 2 == 0))
        def _():
            _do_scatter(sdma1, ddma1, None)

        @pl.when((fcf >= 2) & (fcf % 2 == 0))
        def _():
            pltpu.make_async_copy(ones_v, cnt_sh.at[ddma0], semc0).wait()

        @pl.when((fcf >= 2) & (fcf % 2 == 1))
        def _():
            pltpu.make_async_copy(ones_v, cnt_sh.at[ddma1], semc1).wait()

        offf = off_ref[0]

        @pl.when(offf > 0)
        def _():
            for k in range(FLUSH // LANES):
                sl = pl.ds(k * LANES, LANES)
                posv = lax.iota(jnp.int32, LANES) + (k * LANES)
                keep = posv < offf
                sdma0[sl] = src_buf[sl]
                ddma0[sl] = jnp.where(
                    keep, dst_buf[sl], jnp.full((LANES,), DUMMY, jnp.int32))
            pltpu.async_copy(x_hbm.at[sdma0], rows_v, semg).wait()
            pltpu.sync_copy(rows_v, acc.at[ddma0], add=True)
            pltpu.sync_copy(ones_v, cnt_sh.at[ddma0], add=True)

        plsc.subcore_barrier()

        @pl.when(s < NSUB - 1)
        def _():
            pltpu.sync_copy(acc.at[pl.ds(row0, ROW_CHUNK)],
                            sum_hbm.at[pl.ds(base + row0, ROW_CHUNK)])

        @pl.when(s == NSUB - 1)
        def _():
            pltpu.sync_copy(acc.at[pl.ds(row0, ROW_LAST)],
                            sum_hbm.at[pl.ds(base + row0, ROW_LAST)])

        @pl.when(s < NSUB - 1)
        def _():
            pltpu.sync_copy(cnt_sh.at[pl.ds(s * CNT_CHUNK, CNT_CHUNK)], cbuf)
            pltpu.sync_copy(cbuf,
                            cnt_hbm.at[pl.ds(base + s * CNT_CHUNK, CNT_CHUNK)])

        @pl.when(s == NSUB - 1)
        def _():
            pltpu.sync_copy(cnt_sh.at[pl.ds(s * CNT_CHUNK, CNT_LAST)],
                            cbuf.at[pl.ds(0, CNT_LAST)])
            pltpu.sync_copy(cbuf.at[pl.ds(0, CNT_LAST)],
                            cnt_hbm.at[pl.ds(base + s * CNT_CHUNK, CNT_LAST)])

        plsc.subcore_barrier()


def _sc_segsum(x, src, dst):
    mesh = plsc.VectorSubcoreMesh(core_axis_name="c", subcore_axis_name="s")
    f = pl.kernel(
        _segsum_body,
        out_type=(jax.ShapeDtypeStruct((NPAD, D), jnp.float32),
                  jax.ShapeDtypeStruct((NPAD,), jnp.float32)),
        mesh=mesh,
        compiler_params=dataclasses.replace(
            pltpu.CompilerParams(), needs_layout_passes=False),
        scratch_types=[
            pltpu.VMEM((FLUSH, D), jnp.float32),
            pltpu.VMEM((BLKE,), jnp.int32),
            pltpu.VMEM((BLKE,), jnp.int32),
            pltpu.VMEM((BLKE,), jnp.int32),
            pltpu.VMEM((BLKE,), jnp.int32),
            pltpu.VMEM((CAP,), jnp.int32),
            pltpu.VMEM((CAP,), jnp.int32),
            pltpu.VMEM((FLUSH,), jnp.int32),
            pltpu.VMEM((FLUSH,), jnp.int32),
            pltpu.VMEM((FLUSH,), jnp.int32),
            pltpu.VMEM((FLUSH,), jnp.int32),
            pltpu.VMEM((CNT_CHUNK,), jnp.float32),
            pltpu.VMEM((FLUSH,), jnp.float32),
            pltpu.VMEM((CNT_CHUNK,), jnp.float32),
            pltpu.SMEM((2,), jnp.int32),
            pltpu.VMEM_SHARED((ACC_ROWS, D), jnp.float32),
            pltpu.VMEM_SHARED((ACC_ROWS,), jnp.float32),
            pltpu.SemaphoreType.DMA,
            pltpu.SemaphoreType.DMA,
            pltpu.SemaphoreType.DMA,
            pltpu.SemaphoreType.DMA,
            pltpu.SemaphoreType.DMA,
            pltpu.SemaphoreType.DMA,
            pltpu.SemaphoreType.DMA,
        ],
    )
    return f(x, src, dst)


_BLKR = 5000


def _linr_body(x_ref, wr_ref, o_ref):
    o_ref[...] = lax.dot_general(x_ref[...], wr_ref[...],
                                 (((1,), (0,)), ((), ())),
                                 precision=lax.Precision.HIGHEST,
                                 preferred_element_type=jnp.float32)


def _tc_linr(x_dst, wr):
    return pl.pallas_call(
        _linr_body,
        grid=(N // _BLKR,),
        in_specs=[
            pl.BlockSpec((_BLKR, D), lambda i: (i, 0)),
            pl.BlockSpec((D, D), lambda i: (0, 0)),
        ],
        out_specs=pl.BlockSpec((_BLKR, D), lambda i: (i, 0)),
        out_shape=jax.ShapeDtypeStruct((N, D), jnp.float32),
    )(x_dst, wr)


def _finish_body(sum_ref, rcp_ref, lr_ref, wl_ref, b_ref, o_ref):
    agg = sum_ref[...] * rcp_ref[...]
    y = lax.dot_general(agg, wl_ref[...], (((1,), (0,)), ((), ())),
                        precision=lax.Precision.HIGHEST,
                        preferred_element_type=jnp.float32)
    o_ref[...] = jnp.maximum(y + lr_ref[...] + b_ref[...], 0.0)


def _tc_finish(sum_pad, rcp2, linr, wl, bb):
    return pl.pallas_call(
        _finish_body,
        grid=(N // _BLKR,),
        in_specs=[
            pl.BlockSpec((_BLKR, D), lambda i: (i, 0)),
            pl.BlockSpec((_BLKR, 1), lambda i: (i, 0)),
            pl.BlockSpec((_BLKR, D), lambda i: (i, 0)),
            pl.BlockSpec((D, D), lambda i: (0, 0)),
            pl.BlockSpec((1, D), lambda i: (0, 0)),
        ],
        out_specs=pl.BlockSpec((_BLKR, D), lambda i: (i, 0)),
        out_shape=jax.ShapeDtypeStruct((N, D), jnp.float32),
    )(sum_pad, rcp2, linr, wl, bb)


def _pad_edges(edge):
    src = jnp.concatenate(
        [edge[0].astype(jnp.int32), jnp.zeros((EPAD - E_RAW,), jnp.int32)])
    dst = jnp.concatenate(
        [edge[1].astype(jnp.int32),
         jnp.full((EPAD - E_RAW,), SENTINEL, jnp.int32)])
    return src, dst


def kernel(x_user, x_event, edge_e2u, edge_u2e,
           Wl_u0, bl_u0, Wr_u0, gamma_u0, beta_u0,
           Wl_e0, bl_e0, Wr_e0, gamma_e0, beta_e0,
           Wl_u1, bl_u1, Wr_u1, gamma_u1, beta_u1,
           Wl_e1, bl_e1, Wr_e1, gamma_e1, beta_e1):
    se2u, de2u = _pad_edges(edge_e2u)
    su2e, du2e = _pad_edges(edge_u2e)
    params = {
        "u": [(Wl_u0, bl_u0, Wr_u0, gamma_u0, beta_u0),
              (Wl_u1, bl_u1, Wr_u1, gamma_u1, beta_u1)],
        "e": [(Wl_e0, bl_e0, Wr_e0, gamma_e0, beta_e0),
              (Wl_e1, bl_e1, Wr_e1, gamma_e1, beta_e1)],
    }
    def _prep(Wl, bl, Wr, gamma, beta):
        scale = gamma * _BN_SCALE
        return (Wl.T * scale[None, :], Wr.T * scale[None, :],
                (bl * scale + beta).reshape(1, D))

    xu, xe = x_user, x_event
    rcp_u = rcp_e = None
    for i in range(2):
        wl, wr, bb = _prep(*params["u"][i])
        lr = _tc_linr(xu, wr)
        su, cu = _sc_segsum(xe, se2u, de2u)
        if rcp_u is None:
            rcp_u = (1.0 / jnp.maximum(cu, 1.0)).reshape(-1, 1)
        xu = _tc_finish(su, rcp_u, lr, wl, bb)
        wl, wr, bb = _prep(*params["e"][i])
        lr = _tc_linr(xe, wr)
        se, ce = _sc_segsum(xu, su2e, du2e)
        if rcp_e is None:
            rcp_e = (1.0 / jnp.maximum(ce, 1.0)).reshape(-1, 1)
        xe = _tc_finish(se, rcp_e, lr, wl, bb)
    return (xu, xe)


# speedup vs baseline: 12.7065x; 1.2948x over previous
import dataclasses
import math

import jax
import jax.numpy as jnp
from jax import lax
from jax.experimental import pallas as pl
from jax.experimental.pallas import tpu as pltpu
from jax.experimental.pallas import tpu_sc as plsc

D = 128
N = 50000
E_RAW = 600000
LANES = 16
NSUB = 16

BLKE = 1184
NBLK = 32
EPAD = BLKE * NSUB * NBLK
NPASS = 3
FLUSH = 192
ZROWS = 128
CAP = 208
REG = 8344
NPAD = 2 * NPASS * REG
ACC_ROWS = 8352
DUMMY = REG
SENTINEL = 1 << 30
ROW_CHUNK = 528
ROW_LAST = REG - 15 * ROW_CHUNK
CNT_CHUNK = 528
CNT_LAST = REG - 15 * CNT_CHUNK

_BN_SCALE = 1.0 / math.sqrt(1.0 + 1e-5)


def _segsum_body(x_hbm, src_hbm, dst_hbm, sum_hbm, cnt_hbm,
                 rows0, rows1, bs0, bd0, bs1, bd1, src_buf, dst_buf,
                 sdma0, ddma0, sdma1, ddma1, zbuf, ones_v, cbuf, off_ref,
                 acc, cnt_sh,
                 semg0, semg1, sema0, sema1, semc0, semc1,
                 sembs0, sembd0, sembs1, sembd1):
    c = lax.axis_index("c")
    s = lax.axis_index("s")
    z16f = jnp.zeros((LANES,), jnp.float32)
    z16i = jnp.zeros((LANES,), jnp.int32)
    one16 = jnp.ones((LANES,), jnp.float32)
    row0 = s * ROW_CHUNK

    @pl.loop(0, CNT_CHUNK // LANES)
    def _(i):
        zbuf[pl.ds(i * LANES, LANES)] = z16f

    for k in range(FLUSH // LANES):
        ones_v[pl.ds(k * LANES, LANES)] = one16

    BUFS = ((sdma0, ddma0, rows0, semg0, sema0, semc0),
            (sdma1, ddma1, rows1, semg1, sema1, semc1))

    for p in range(NPASS):
        base = (2 * p + c) * REG

        @pl.loop(0, ZROWS)
        def _(r):
            for k in range(D // LANES):
                rows0[r, pl.ds(k * LANES, LANES)] = z16f

        zstage = rows0.at[pl.ds(0, ZROWS)]
        for k in range(3):
            pltpu.sync_copy(zstage, acc.at[pl.ds(row0 + k * ZROWS, ZROWS)])

        @pl.when(s < NSUB - 1)
        def _():
            pltpu.sync_copy(zstage, acc.at[pl.ds(row0 + 3 * ZROWS, ZROWS)])
            pltpu.sync_copy(rows0.at[pl.ds(0, ROW_CHUNK - 4 * ZROWS)],
                            acc.at[pl.ds(row0 + 4 * ZROWS,
                                         ROW_CHUNK - 4 * ZROWS)])

        @pl.when(s == NSUB - 1)
        def _():
            pltpu.sync_copy(rows0.at[pl.ds(0, ROW_LAST - 3 * ZROWS)],
                            acc.at[pl.ds(row0 + 3 * ZROWS,
                                         ROW_LAST - 3 * ZROWS)])

        @pl.when(s < NSUB - 1)
        def _():
            pltpu.sync_copy(zbuf, cnt_sh.at[pl.ds(s * CNT_CHUNK, CNT_CHUNK)])

        @pl.when(s == NSUB - 1)
        def _():
            pltpu.sync_copy(zbuf.at[pl.ds(0, CNT_LAST)],
                            cnt_sh.at[pl.ds(s * CNT_CHUNK, CNT_LAST)])

        for k in range(CAP // LANES):
            src_buf[pl.ds(k * LANES, LANES)] = z16i
            dst_buf[pl.ds(k * LANES, LANES)] = z16i
        off_ref[0] = 0
        off_ref[1] = 0
        plsc.subcore_barrier()

        def _blk_start(b, bs, bd, ss, sd):
            e0 = (b * NSUB + s) * BLKE
            pltpu.async_copy(src_hbm.at[pl.ds(e0, BLKE)], bs, ss)
            pltpu.async_copy(dst_hbm.at[pl.ds(e0, BLKE)], bd, sd)

        def _blk_wait(b, bs, bd, ss, sd):
            e0 = (b * NSUB + s) * BLKE
            pltpu.make_async_copy(src_hbm.at[pl.ds(e0, BLKE)], bs, ss).wait()
            pltpu.make_async_copy(dst_hbm.at[pl.ds(e0, BLKE)], bd, sd).wait()

        def _flush(q, fc):
            sd, dd, rows, semg, sema, semc = BUFS[q]
            osd, odd, orows, osemg, osema, osemc = BUFS[1 - q]

            @pl.when(fc >= 2)
            def _():
                pltpu.make_async_copy(rows, acc.at[dd], sema).wait()
                pltpu.make_async_copy(ones_v, cnt_sh.at[dd], semc).wait()
            for k in range(FLUSH // LANES):
                sl = pl.ds(k * LANES, LANES)
                sd[sl] = src_buf[sl]
                dd[sl] = dst_buf[sl]
            pltpu.async_copy(x_hbm.at[sd], rows, semg)

            @pl.when(fc >= 1)
            def _():
                pltpu.make_async_copy(x_hbm.at[osd], orows, osemg).wait()
                pltpu.async_copy(orows, acc.at[odd], osema, add=True)
                pltpu.async_copy(ones_v, cnt_sh.at[odd], osemc, add=True)

        def _chunks(bs, bd):
            @pl.loop(0, BLKE // LANES)
            def _(k):
                sl = pl.ds(k * LANES, LANES)
                d16 = bd[sl]
                s16 = bs[sl]
                m = (d16 >= base) & (d16 < base + REG)
                mi = m.astype(jnp.int32)
                off0 = off_ref[0]
                pos = jnp.cumsum(mi) - mi + off0
                plsc.store_scatter(src_buf, [pos], s16, mask=m)
                plsc.store_scatter(dst_buf, [pos], d16 - base, mask=m)
                off1 = off0 + jnp.sum(mi)
                off_ref[0] = off1

                @pl.when(off1 >= FLUSH)
                def _():
                    fc = off_ref[1]

                    @pl.when(fc % 2 == 0)
                    def _():
                        _flush(0, fc)

                    @pl.when(fc % 2 == 1)
                    def _():
                        _flush(1, fc)
                    src_buf[pl.ds(0, LANES)] = src_buf[pl.ds(FLUSH, LANES)]
                    dst_buf[pl.ds(0, LANES)] = dst_buf[pl.ds(FLUSH, LANES)]
                    off_ref[0] = off1 - FLUSH
                    off_ref[1] = fc + 1

        _blk_start(0, bs0, bd0, sembs0, sembd0)

        @pl.loop(0, NBLK // 2)
        def _(t):
            b0 = 2 * t
            _blk_wait(b0, bs0, bd0, sembs0, sembd0)
            _blk_start(b0 + 1, bs1, bd1, sembs1, sembd1)
            _chunks(bs0, bd0)
            _blk_wait(b0 + 1, bs1, bd1, sembs1, sembd1)

            @pl.when(t < NBLK // 2 - 1)
            def _():
                _blk_start(b0 + 2, bs0, bd0, sembs0, sembd0)
            _chunks(bs1, bd1)

        fcf = off_ref[1]

        def _wait_adds(q):
            sd, dd, rows, semg, sema, semc = BUFS[q]
            pltpu.make_async_copy(rows, acc.at[dd], sema).wait()
            pltpu.make_async_copy(ones_v, cnt_sh.at[dd], semc).wait()

        def _last_scatter(q):
            sd, dd, rows, semg, sema, semc = BUFS[q]
            pltpu.make_async_copy(x_hbm.at[sd], rows, semg).wait()
            pltpu.sync_copy(rows, acc.at[dd], add=True)
            pltpu.sync_copy(ones_v, cnt_sh.at[dd], add=True)

        @pl.when((fcf >= 2) & (fcf % 2 == 0))
        def _():
            _wait_adds(0)

        @pl.when((fcf >= 2) & (fcf % 2 == 1))
        def _():
            _wait_adds(1)

        @pl.when((fcf >= 1) & (fcf % 2 == 1))
        def _():
            _last_scatter(0)

        @pl.when((fcf >= 1) & (fcf % 2 == 0))
        def _():
            _last_scatter(1)

        offf = off_ref[0]

        @pl.when(offf > 0)
        def _():
            for k in range(FLUSH // LANES):
                sl = pl.ds(k * LANES, LANES)
                posv = lax.iota(jnp.int32, LANES) + (k * LANES)
                keep = posv < offf
                sdma0[sl] = src_buf[sl]
                ddma0[sl] = jnp.where(
                    keep, dst_buf[sl], jnp.full((LANES,), DUMMY, jnp.int32))
            pltpu.async_copy(x_hbm.at[sdma0], rows0, semg0).wait()
            pltpu.sync_copy(rows0, acc.at[ddma0], add=True)
            pltpu.sync_copy(ones_v, cnt_sh.at[ddma0], add=True)

        plsc.subcore_barrier()

        @pl.when(s < NSUB - 1)
        def _():
            pltpu.sync_copy(acc.at[pl.ds(row0, ROW_CHUNK)],
                            sum_hbm.at[pl.ds(base + row0, ROW_CHUNK)])

        @pl.when(s == NSUB - 1)
        def _():
            pltpu.sync_copy(acc.at[pl.ds(row0, ROW_LAST)],
                            sum_hbm.at[pl.ds(base + row0, ROW_LAST)])

        @pl.when(s < NSUB - 1)
        def _():
            pltpu.sync_copy(cnt_sh.at[pl.ds(s * CNT_CHUNK, CNT_CHUNK)], cbuf)
            pltpu.sync_copy(cbuf,
                            cnt_hbm.at[pl.ds(base + s * CNT_CHUNK, CNT_CHUNK)])

        @pl.when(s == NSUB - 1)
        def _():
            pltpu.sync_copy(cnt_sh.at[pl.ds(s * CNT_CHUNK, CNT_LAST)],
                            cbuf.at[pl.ds(0, CNT_LAST)])
            pltpu.sync_copy(cbuf.at[pl.ds(0, CNT_LAST)],
                            cnt_hbm.at[pl.ds(base + s * CNT_CHUNK, CNT_LAST)])

        plsc.subcore_barrier()


def _sc_segsum(x, src, dst):
    mesh = plsc.VectorSubcoreMesh(core_axis_name="c", subcore_axis_name="s")
    f = pl.kernel(
        _segsum_body,
        out_type=(jax.ShapeDtypeStruct((NPAD, D), jnp.float32),
                  jax.ShapeDtypeStruct((NPAD,), jnp.float32)),
        mesh=mesh,
        compiler_params=dataclasses.replace(
            pltpu.CompilerParams(), needs_layout_passes=False),
        scratch_types=[
            pltpu.VMEM((FLUSH, D), jnp.float32),
            pltpu.VMEM((FLUSH, D), jnp.float32),
            pltpu.VMEM((BLKE,), jnp.int32),
            pltpu.VMEM((BLKE,), jnp.int32),
            pltpu.VMEM((BLKE,), jnp.int32),
            pltpu.VMEM((BLKE,), jnp.int32),
            pltpu.VMEM((CAP,), jnp.int32),
            pltpu.VMEM((CAP,), jnp.int32),
            pltpu.VMEM((FLUSH,), jnp.int32),
            pltpu.VMEM((FLUSH,), jnp.int32),
            pltpu.VMEM((FLUSH,), jnp.int32),
            pltpu.VMEM((FLUSH,), jnp.int32),
            pltpu.VMEM((CNT_CHUNK,), jnp.float32),
            pltpu.VMEM((FLUSH,), jnp.float32),
            pltpu.VMEM((CNT_CHUNK,), jnp.float32),
            pltpu.SMEM((2,), jnp.int32),
            pltpu.VMEM_SHARED((ACC_ROWS, D), jnp.float32),
            pltpu.VMEM_SHARED((ACC_ROWS,), jnp.float32),
            pltpu.SemaphoreType.DMA,
            pltpu.SemaphoreType.DMA,
            pltpu.SemaphoreType.DMA,
            pltpu.SemaphoreType.DMA,
            pltpu.SemaphoreType.DMA,
            pltpu.SemaphoreType.DMA,
            pltpu.SemaphoreType.DMA,
            pltpu.SemaphoreType.DMA,
            pltpu.SemaphoreType.DMA,
            pltpu.SemaphoreType.DMA,
        ],
    )
    return f(x, src, dst)


_BLKR = 5000


def _linr_body(x_ref, wr_ref, o_ref):
    o_ref[...] = lax.dot_general(x_ref[...], wr_ref[...],
                                 (((1,), (0,)), ((), ())),
                                 precision=lax.Precision.HIGHEST,
                                 preferred_element_type=jnp.float32)


def _tc_linr(x_dst, wr):
    return pl.pallas_call(
        _linr_body,
        grid=(N // _BLKR,),
        in_specs=[
            pl.BlockSpec((_BLKR, D), lambda i: (i, 0)),
            pl.BlockSpec((D, D), lambda i: (0, 0)),
        ],
        out_specs=pl.BlockSpec((_BLKR, D), lambda i: (i, 0)),
        out_shape=jax.ShapeDtypeStruct((N, D), jnp.float32),
    )(x_dst, wr)


def _finish_body(sum_ref, rcp_ref, lr_ref, wl_ref, b_ref, o_ref):
    agg = sum_ref[...] * rcp_ref[...]
    y = lax.dot_general(agg, wl_ref[...], (((1,), (0,)), ((), ())),
                        precision=lax.Precision.HIGHEST,
                        preferred_element_type=jnp.float32)
    o_ref[...] = jnp.maximum(y + lr_ref[...] + b_ref[...], 0.0)


def _tc_finish(sum_pad, rcp2, linr, wl, bb):
    return pl.pallas_call(
        _finish_body,
        grid=(N // _BLKR,),
        in_specs=[
            pl.BlockSpec((_BLKR, D), lambda i: (i, 0)),
            pl.BlockSpec((_BLKR, 1), lambda i: (i, 0)),
            pl.BlockSpec((_BLKR, D), lambda i: (i, 0)),
            pl.BlockSpec((D, D), lambda i: (0, 0)),
            pl.BlockSpec((1, D), lambda i: (0, 0)),
        ],
        out_specs=pl.BlockSpec((_BLKR, D), lambda i: (i, 0)),
        out_shape=jax.ShapeDtypeStruct((N, D), jnp.float32),
    )(sum_pad, rcp2, linr, wl, bb)


def _pad_edges(edge):
    src = jnp.concatenate(
        [edge[0].astype(jnp.int32), jnp.zeros((EPAD - E_RAW,), jnp.int32)])
    dst = jnp.concatenate(
        [edge[1].astype(jnp.int32),
         jnp.full((EPAD - E_RAW,), SENTINEL, jnp.int32)])
    return src, dst


def kernel(x_user, x_event, edge_e2u, edge_u2e,
           Wl_u0, bl_u0, Wr_u0, gamma_u0, beta_u0,
           Wl_e0, bl_e0, Wr_e0, gamma_e0, beta_e0,
           Wl_u1, bl_u1, Wr_u1, gamma_u1, beta_u1,
           Wl_e1, bl_e1, Wr_e1, gamma_e1, beta_e1):
    se2u, de2u = _pad_edges(edge_e2u)
    su2e, du2e = _pad_edges(edge_u2e)
    params = {
        "u": [(Wl_u0, bl_u0, Wr_u0, gamma_u0, beta_u0),
              (Wl_u1, bl_u1, Wr_u1, gamma_u1, beta_u1)],
        "e": [(Wl_e0, bl_e0, Wr_e0, gamma_e0, beta_e0),
              (Wl_e1, bl_e1, Wr_e1, gamma_e1, beta_e1)],
    }

    def _prep(Wl, bl, Wr, gamma, beta):
        scale = gamma * _BN_SCALE
        return (Wl.T * scale[None, :], Wr.T * scale[None, :],
                (bl * scale + beta).reshape(1, D))

    xu, xe = x_user, x_event
    rcp_u = rcp_e = None
    for i in range(2):
        wl, wr, bb = _prep(*params["u"][i])
        lr = _tc_linr(xu, wr)
        su, cu = _sc_segsum(xe, se2u, de2u)
        if rcp_u is None:
            rcp_u = (1.0 / jnp.maximum(cu, 1.0)).reshape(-1, 1)
        xu = _tc_finish(su, rcp_u, lr, wl, bb)
        wl, wr, bb = _prep(*params["e"][i])
        lr = _tc_linr(xe, wr)
        se, ce = _sc_segsum(xu, su2e, du2e)
        if rcp_e is None:
            rcp_e = (1.0 / jnp.maximum(ce, 1.0)).reshape(-1, 1)
        xe = _tc_finish(se, rcp_e, lr, wl, bb)
    return (xu, xe)


# speedup vs baseline: 13.0275x; 1.0253x over previous
import dataclasses
import math

import jax
import jax.numpy as jnp
from jax import lax
from jax.experimental import pallas as pl
from jax.experimental.pallas import tpu as pltpu
from jax.experimental.pallas import tpu_sc as plsc

D = 128
N = 50000
E_RAW = 600000
LANES = 16
NSUB = 16

BLKE = 1184
NBLK = 32
EPAD = BLKE * NSUB * NBLK
NPASS = 3
FLUSH = 192
ZROWS = 128
CAP = 208
REG = 8344
NPAD = 2 * NPASS * REG
ACC_ROWS = 8352
DUMMY = REG
SENTINEL = 1 << 30
ROW_CHUNK = 528
ROW_LAST = REG - 15 * ROW_CHUNK
CNT_CHUNK = 528
CNT_LAST = REG - 15 * CNT_CHUNK

_BN_SCALE = 1.0 / math.sqrt(1.0 + 1e-5)


def _segsum_body(x_hbm, src_hbm, dst_hbm, sum_hbm, cnt_hbm,
                 rows0, rows1, bs0, bd0, bs1, bd1, src_buf, dst_buf,
                 sdma0, ddma0, sdma1, ddma1, zbuf, ones_v, cbuf, off_ref,
                 acc, cnt_sh,
                 semg0, semg1, sema0, sema1, semc0, semc1,
                 sembs0, sembd0, sembs1, sembd1):
    c = lax.axis_index("c")
    s = lax.axis_index("s")
    z16f = jnp.zeros((LANES,), jnp.float32)
    z16i = jnp.zeros((LANES,), jnp.int32)
    one16 = jnp.ones((LANES,), jnp.float32)
    row0 = s * ROW_CHUNK

    @pl.loop(0, CNT_CHUNK // LANES)
    def _(i):
        zbuf[pl.ds(i * LANES, LANES)] = z16f

    for k in range(FLUSH // LANES):
        ones_v[pl.ds(k * LANES, LANES)] = one16

    BUFS = ((sdma0, ddma0, rows0, semg0, sema0, semc0),
            (sdma1, ddma1, rows1, semg1, sema1, semc1))

    for p in range(NPASS):
        base = (2 * p + c) * REG

        @pl.loop(0, ZROWS)
        def _(r):
            for k in range(D // LANES):
                rows0[r, pl.ds(k * LANES, LANES)] = z16f

        zstage = rows0.at[pl.ds(0, ZROWS)]
        for k in range(3):
            pltpu.sync_copy(zstage, acc.at[pl.ds(row0 + k * ZROWS, ZROWS)])

        @pl.when(s < NSUB - 1)
        def _():
            pltpu.sync_copy(zstage, acc.at[pl.ds(row0 + 3 * ZROWS, ZROWS)])
            pltpu.sync_copy(rows0.at[pl.ds(0, ROW_CHUNK - 4 * ZROWS)],
                            acc.at[pl.ds(row0 + 4 * ZROWS,
                                         ROW_CHUNK - 4 * ZROWS)])

        @pl.when(s == NSUB - 1)
        def _():
            pltpu.sync_copy(rows0.at[pl.ds(0, ROW_LAST - 3 * ZROWS)],
                            acc.at[pl.ds(row0 + 3 * ZROWS,
                                         ROW_LAST - 3 * ZROWS)])

        @pl.when(s < NSUB - 1)
        def _():
            pltpu.sync_copy(zbuf, cnt_sh.at[pl.ds(s * CNT_CHUNK, CNT_CHUNK)])

        @pl.when(s == NSUB - 1)
        def _():
            pltpu.sync_copy(zbuf.at[pl.ds(0, CNT_LAST)],
                            cnt_sh.at[pl.ds(s * CNT_CHUNK, CNT_LAST)])

        for k in range(CAP // LANES):
            src_buf[pl.ds(k * LANES, LANES)] = z16i
            dst_buf[pl.ds(k * LANES, LANES)] = z16i
        off_ref[0] = 0
        off_ref[1] = 0
        plsc.subcore_barrier()

        def _blk_start(b, bs, bd, ss, sd):
            e0 = (b * NSUB + s) * BLKE
            pltpu.async_copy(src_hbm.at[pl.ds(e0, BLKE)], bs, ss)
            pltpu.async_copy(dst_hbm.at[pl.ds(e0, BLKE)], bd, sd)

        def _blk_wait(b, bs, bd, ss, sd):
            e0 = (b * NSUB + s) * BLKE
            pltpu.make_async_copy(src_hbm.at[pl.ds(e0, BLKE)], bs, ss).wait()
            pltpu.make_async_copy(dst_hbm.at[pl.ds(e0, BLKE)], bd, sd).wait()

        def _flush(q, fc):
            sd, dd, rows, semg, sema, semc = BUFS[q]
            osd, odd, orows, osemg, osema, osemc = BUFS[1 - q]

            @pl.when(fc >= 2)
            def _():
                pltpu.make_async_copy(rows, acc.at[dd], sema).wait()
                pltpu.make_async_copy(ones_v, cnt_sh.at[dd], semc).wait()
            for k in range(FLUSH // LANES):
                sl = pl.ds(k * LANES, LANES)
                sd[sl] = src_buf[sl]
                dd[sl] = dst_buf[sl]
            pltpu.async_copy(x_hbm.at[sd], rows, semg)

            @pl.when(fc >= 1)
            def _():
                pltpu.make_async_copy(x_hbm.at[osd], orows, osemg).wait()
                pltpu.async_copy(orows, acc.at[odd], osema, add=True)
                pltpu.async_copy(ones_v, cnt_sh.at[odd], osemc, add=True)

        def _chunks(bs, bd):
            @pl.loop(0, BLKE // LANES, unroll=2)
            def _(k):
                sl = pl.ds(k * LANES, LANES)
                d16 = bd[sl]
                s16 = bs[sl]
                m = (d16 >= base) & (d16 < base + REG)
                mi = m.astype(jnp.int32)
                off0 = off_ref[0]
                cum = jnp.cumsum(mi)
                pos = cum - mi + off0
                plsc.store_scatter(src_buf, [pos], s16, mask=m)
                plsc.store_scatter(dst_buf, [pos], d16 - base, mask=m)
                off1 = off0 + cum[LANES - 1]
                off_ref[0] = off1

                @pl.when(off1 >= FLUSH)
                def _():
                    fc = off_ref[1]

                    @pl.when(fc % 2 == 0)
                    def _():
                        _flush(0, fc)

                    @pl.when(fc % 2 == 1)
                    def _():
                        _flush(1, fc)
                    src_buf[pl.ds(0, LANES)] = src_buf[pl.ds(FLUSH, LANES)]
                    dst_buf[pl.ds(0, LANES)] = dst_buf[pl.ds(FLUSH, LANES)]
                    off_ref[0] = off1 - FLUSH
                    off_ref[1] = fc + 1

        _blk_start(0, bs0, bd0, sembs0, sembd0)

        @pl.loop(0, NBLK // 2)
        def _(t):
            b0 = 2 * t
            _blk_wait(b0, bs0, bd0, sembs0, sembd0)
            _blk_start(b0 + 1, bs1, bd1, sembs1, sembd1)
            _chunks(bs0, bd0)
            _blk_wait(b0 + 1, bs1, bd1, sembs1, sembd1)

            @pl.when(t < NBLK // 2 - 1)
            def _():
                _blk_start(b0 + 2, bs0, bd0, sembs0, sembd0)
            _chunks(bs1, bd1)

        fcf = off_ref[1]

        def _wait_adds(q):
            sd, dd, rows, semg, sema, semc = BUFS[q]
            pltpu.make_async_copy(rows, acc.at[dd], sema).wait()
            pltpu.make_async_copy(ones_v, cnt_sh.at[dd], semc).wait()

        def _last_scatter(q):
            sd, dd, rows, semg, sema, semc = BUFS[q]
            pltpu.make_async_copy(x_hbm.at[sd], rows, semg).wait()
            pltpu.sync_copy(rows, acc.at[dd], add=True)
            pltpu.sync_copy(ones_v, cnt_sh.at[dd], add=True)

        @pl.when((fcf >= 2) & (fcf % 2 == 0))
        def _():
            _wait_adds(0)

        @pl.when((fcf >= 2) & (fcf % 2 == 1))
        def _():
            _wait_adds(1)

        @pl.when((fcf >= 1) & (fcf % 2 == 1))
        def _():
            _last_scatter(0)

        @pl.when((fcf >= 1) & (fcf % 2 == 0))
        def _():
            _last_scatter(1)

        offf = off_ref[0]

        @pl.when(offf > 0)
        def _():
            for k in range(FLUSH // LANES):
                sl = pl.ds(k * LANES, LANES)
                posv = lax.iota(jnp.int32, LANES) + (k * LANES)
                keep = posv < offf
                sdma0[sl] = src_buf[sl]
                ddma0[sl] = jnp.where(
                    keep, dst_buf[sl], jnp.full((LANES,), DUMMY, jnp.int32))
            pltpu.async_copy(x_hbm.at[sdma0], rows0, semg0).wait()
            pltpu.sync_copy(rows0, acc.at[ddma0], add=True)
            pltpu.sync_copy(ones_v, cnt_sh.at[ddma0], add=True)

        plsc.subcore_barrier()

        @pl.when(s < NSUB - 1)
        def _():
            pltpu.sync_copy(acc.at[pl.ds(row0, ROW_CHUNK)],
                            sum_hbm.at[pl.ds(base + row0, ROW_CHUNK)])

        @pl.when(s == NSUB - 1)
        def _():
            pltpu.sync_copy(acc.at[pl.ds(row0, ROW_LAST)],
                            sum_hbm.at[pl.ds(base + row0, ROW_LAST)])

        @pl.when(s < NSUB - 1)
        def _():
            pltpu.sync_copy(cnt_sh.at[pl.ds(s * CNT_CHUNK, CNT_CHUNK)], cbuf)
            pltpu.sync_copy(cbuf,
                            cnt_hbm.at[pl.ds(base + s * CNT_CHUNK, CNT_CHUNK)])

        @pl.when(s == NSUB - 1)
        def _():
            pltpu.sync_copy(cnt_sh.at[pl.ds(s * CNT_CHUNK, CNT_LAST)],
                            cbuf.at[pl.ds(0, CNT_LAST)])
            pltpu.sync_copy(cbuf.at[pl.ds(0, CNT_LAST)],
                            cnt_hbm.at[pl.ds(base + s * CNT_CHUNK, CNT_LAST)])

        plsc.subcore_barrier()


def _sc_segsum(x, src, dst):
    mesh = plsc.VectorSubcoreMesh(core_axis_name="c", subcore_axis_name="s")
    f = pl.kernel(
        _segsum_body,
        out_type=(jax.ShapeDtypeStruct((NPAD, D), jnp.float32),
                  jax.ShapeDtypeStruct((NPAD,), jnp.float32)),
        mesh=mesh,
        compiler_params=dataclasses.replace(
            pltpu.CompilerParams(), needs_layout_passes=False),
        scratch_types=[
            pltpu.VMEM((FLUSH, D), jnp.float32),
            pltpu.VMEM((FLUSH, D), jnp.float32),
            pltpu.VMEM((BLKE,), jnp.int32),
            pltpu.VMEM((BLKE,), jnp.int32),
            pltpu.VMEM((BLKE,), jnp.int32),
            pltpu.VMEM((BLKE,), jnp.int32),
            pltpu.VMEM((CAP,), jnp.int32),
            pltpu.VMEM((CAP,), jnp.int32),
            pltpu.VMEM((FLUSH,), jnp.int32),
            pltpu.VMEM((FLUSH,), jnp.int32),
            pltpu.VMEM((FLUSH,), jnp.int32),
            pltpu.VMEM((FLUSH,), jnp.int32),
            pltpu.VMEM((CNT_CHUNK,), jnp.float32),
            pltpu.VMEM((FLUSH,), jnp.float32),
            pltpu.VMEM((CNT_CHUNK,), jnp.float32),
            pltpu.SMEM((2,), jnp.int32),
            pltpu.VMEM_SHARED((ACC_ROWS, D), jnp.float32),
            pltpu.VMEM_SHARED((ACC_ROWS,), jnp.float32),
            pltpu.SemaphoreType.DMA,
            pltpu.SemaphoreType.DMA,
            pltpu.SemaphoreType.DMA,
            pltpu.SemaphoreType.DMA,
            pltpu.SemaphoreType.DMA,
            pltpu.SemaphoreType.DMA,
            pltpu.SemaphoreType.DMA,
            pltpu.SemaphoreType.DMA,
            pltpu.SemaphoreType.DMA,
            pltpu.SemaphoreType.DMA,
        ],
    )
    return f(x, src, dst)


_BLKR = 5000


def _linr_body(x_ref, wr_ref, o_ref):
    o_ref[...] = lax.dot_general(x_ref[...], wr_ref[...],
                                 (((1,), (0,)), ((), ())),
                                 precision=lax.Precision.HIGHEST,
                                 preferred_element_type=jnp.float32)


def _tc_linr(x_dst, wr):
    return pl.pallas_call(
        _linr_body,
        grid=(N // _BLKR,),
        in_specs=[
            pl.BlockSpec((_BLKR, D), lambda i: (i, 0)),
            pl.BlockSpec((D, D), lambda i: (0, 0)),
        ],
        out_specs=pl.BlockSpec((_BLKR, D), lambda i: (i, 0)),
        out_shape=jax.ShapeDtypeStruct((N, D), jnp.float32),
    )(x_dst, wr)


_BLKF = 5000


def _finish_body(sum_ref, rcp_ref, lr_ref, wl_ref, b_ref, o_ref):
    agg = sum_ref[...] * rcp_ref[...]
    y = lax.dot_general(agg, wl_ref[...], (((1,), (0,)), ((), ())),
                        precision=lax.Precision.HIGHEST,
                        preferred_element_type=jnp.float32)
    o_ref[...] = jnp.maximum(y + lr_ref[...] + b_ref[...], 0.0)


def _tc_finish(sum_pad, rcp2, linr, wl, bb):
    return pl.pallas_call(
        _finish_body,
        grid=(N // _BLKF,),
        in_specs=[
            pl.BlockSpec((_BLKF, D), lambda i: (i, 0)),
            pl.BlockSpec((_BLKF, 1), lambda i: (i, 0)),
            pl.BlockSpec((_BLKF, D), lambda i: (i, 0)),
            pl.BlockSpec((D, D), lambda i: (0, 0)),
            pl.BlockSpec((1, D), lambda i: (0, 0)),
        ],
        out_specs=pl.BlockSpec((_BLKF, D), lambda i: (i, 0)),
        out_shape=jax.ShapeDtypeStruct((N, D), jnp.float32),
    )(sum_pad, rcp2, linr, wl, bb)


def _pad_edges(edge):
    src = jnp.concatenate(
        [edge[0].astype(jnp.int32), jnp.zeros((EPAD - E_RAW,), jnp.int32)])
    dst = jnp.concatenate(
        [edge[1].astype(jnp.int32),
         jnp.full((EPAD - E_RAW,), SENTINEL, jnp.int32)])
    return src, dst


def kernel(x_user, x_event, edge_e2u, edge_u2e,
           Wl_u0, bl_u0, Wr_u0, gamma_u0, beta_u0,
           Wl_e0, bl_e0, Wr_e0, gamma_e0, beta_e0,
           Wl_u1, bl_u1, Wr_u1, gamma_u1, beta_u1,
           Wl_e1, bl_e1, Wr_e1, gamma_e1, beta_e1):
    se2u, de2u = _pad_edges(edge_e2u)
    su2e, du2e = _pad_edges(edge_u2e)
    params = {
        "u": [(Wl_u0, bl_u0, Wr_u0, gamma_u0, beta_u0),
              (Wl_u1, bl_u1, Wr_u1, gamma_u1, beta_u1)],
        "e": [(Wl_e0, bl_e0, Wr_e0, gamma_e0, beta_e0),
              (Wl_e1, bl_e1, Wr_e1, gamma_e1, beta_e1)],
    }

    def _prep(Wl, bl, Wr, gamma, beta):
        scale = gamma * _BN_SCALE
        return (Wl.T * scale[None, :], Wr.T * scale[None, :],
                (bl * scale + beta).reshape(1, D))

    xu, xe = x_user, x_event
    rcp_u = rcp_e = None
    for i in range(2):
        wl, wr, bb = _prep(*params["u"][i])
        lr = _tc_linr(xu, wr)
        su, cu = _sc_segsum(xe, se2u, de2u)
        if rcp_u is None:
            rcp_u = (1.0 / jnp.maximum(cu, 1.0)).reshape(-1, 1)
        xu = _tc_finish(su, rcp_u, lr, wl, bb)
        wl, wr, bb = _prep(*params["e"][i])
        lr = _tc_linr(xe, wr)
        se, ce = _sc_segsum(xu, su2e, du2e)
        if rcp_e is None:
            rcp_e = (1.0 / jnp.maximum(ce, 1.0)).reshape(-1, 1)
        xe = _tc_finish(se, rcp_e, lr, wl, bb)
    return (xu, xe)


# speedup vs baseline: 13.5336x; 1.0388x over previous
import dataclasses
import math

import jax
import jax.numpy as jnp
from jax import lax
from jax.experimental import pallas as pl
from jax.experimental.pallas import tpu as pltpu
from jax.experimental.pallas import tpu_sc as plsc

D = 128
N = 50000
E_RAW = 600000
LANES = 16
NSUB = 16

BLKE = 1184
NBLK = 32
EPAD = BLKE * NSUB * NBLK
NPASS = 3
FLUSH = 192
ZROWS = 128
CAP = 208
REG = 8344
NPAD = 2 * NPASS * REG
ACC_ROWS = 8352
DUMMY = REG
SENTINEL = 1 << 30
ROW_CHUNK = 528
ROW_LAST = REG - 15 * ROW_CHUNK
CNT_CHUNK = 528
CNT_LAST = REG - 15 * CNT_CHUNK

_BN_SCALE = 1.0 / math.sqrt(1.0 + 1e-5)


def _segsum_body(x_hbm, src_hbm, dst_hbm, sum_hbm, cnt_hbm,
                 rows0, rows1, bs0, bd0, bs1, bd1, src_buf, dst_buf,
                 sdma0, ddma0, sdma1, ddma1, zbuf, ones_v, cbuf, off_ref,
                 acc, cnt_sh,
                 semg0, semg1, sema0, sema1, semc0, semc1,
                 sembs0, sembd0, sembs1, sembd1):
    c = lax.axis_index("c")
    s = lax.axis_index("s")
    z16f = jnp.zeros((LANES,), jnp.float32)
    z16i = jnp.zeros((LANES,), jnp.int32)
    one16 = jnp.ones((LANES,), jnp.float32)
    row0 = s * ROW_CHUNK

    @pl.loop(0, CNT_CHUNK // LANES)
    def _(i):
        zbuf[pl.ds(i * LANES, LANES)] = z16f

    for k in range(FLUSH // LANES):
        ones_v[pl.ds(k * LANES, LANES)] = one16

    BUFS = ((sdma0, ddma0, rows0, semg0, sema0, semc0),
            (sdma1, ddma1, rows1, semg1, sema1, semc1))

    for p in range(NPASS):
        base = (2 * p + c) * REG

        @pl.loop(0, ZROWS)
        def _(r):
            for k in range(D // LANES):
                rows0[r, pl.ds(k * LANES, LANES)] = z16f

        zstage = rows0.at[pl.ds(0, ZROWS)]
        for k in range(3):
            pltpu.sync_copy(zstage, acc.at[pl.ds(row0 + k * ZROWS, ZROWS)])

        @pl.when(s < NSUB - 1)
        def _():
            pltpu.sync_copy(zstage, acc.at[pl.ds(row0 + 3 * ZROWS, ZROWS)])
            pltpu.sync_copy(rows0.at[pl.ds(0, ROW_CHUNK - 4 * ZROWS)],
                            acc.at[pl.ds(row0 + 4 * ZROWS,
                                         ROW_CHUNK - 4 * ZROWS)])

        @pl.when(s == NSUB - 1)
        def _():
            pltpu.sync_copy(rows0.at[pl.ds(0, ROW_LAST - 3 * ZROWS)],
                            acc.at[pl.ds(row0 + 3 * ZROWS,
                                         ROW_LAST - 3 * ZROWS)])

        @pl.when(s < NSUB - 1)
        def _():
            pltpu.sync_copy(zbuf, cnt_sh.at[pl.ds(s * CNT_CHUNK, CNT_CHUNK)])

        @pl.when(s == NSUB - 1)
        def _():
            pltpu.sync_copy(zbuf.at[pl.ds(0, CNT_LAST)],
                            cnt_sh.at[pl.ds(s * CNT_CHUNK, CNT_LAST)])

        for k in range(CAP // LANES):
            src_buf[pl.ds(k * LANES, LANES)] = z16i
            dst_buf[pl.ds(k * LANES, LANES)] = z16i
        off_ref[0] = 0
        off_ref[1] = 0
        plsc.subcore_barrier()

        def _blk_start(b, bs, bd, ss, sd):
            e0 = (b * NSUB + s) * BLKE
            pltpu.async_copy(src_hbm.at[pl.ds(e0, BLKE)], bs, ss)
            pltpu.async_copy(dst_hbm.at[pl.ds(e0, BLKE)], bd, sd)

        def _blk_wait(b, bs, bd, ss, sd):
            e0 = (b * NSUB + s) * BLKE
            pltpu.make_async_copy(src_hbm.at[pl.ds(e0, BLKE)], bs, ss).wait()
            pltpu.make_async_copy(dst_hbm.at[pl.ds(e0, BLKE)], bd, sd).wait()

        def _flush(q, fc):
            sd, dd, rows, semg, sema, semc = BUFS[q]
            osd, odd, orows, osemg, osema, osemc = BUFS[1 - q]

            @pl.when(fc >= 2)
            def _():
                pltpu.make_async_copy(rows, acc.at[dd], sema).wait()
                pltpu.make_async_copy(ones_v, cnt_sh.at[dd], semc).wait()
            for k in range(FLUSH // LANES):
                sl = pl.ds(k * LANES, LANES)
                sd[sl] = src_buf[sl]
                dd[sl] = dst_buf[sl]
            pltpu.async_copy(x_hbm.at[sd], rows, semg)

            @pl.when(fc >= 1)
            def _():
                pltpu.make_async_copy(x_hbm.at[osd], orows, osemg).wait()
                pltpu.async_copy(orows, acc.at[odd], osema, add=True)
                pltpu.async_copy(ones_v, cnt_sh.at[odd], osemc, add=True)

        def _chunks(bs, bd):
            @pl.loop(0, BLKE // LANES, unroll=2)
            def _(k):
                sl = pl.ds(k * LANES, LANES)
                d16 = bd[sl]
                s16 = bs[sl]
                m = (d16 >= base) & (d16 < base + REG)
                mi = m.astype(jnp.int32)
                off0 = off_ref[0]
                cum = jnp.cumsum(mi)
                pos = cum - mi + off0
                plsc.store_scatter(src_buf, [pos], s16, mask=m)
                plsc.store_scatter(dst_buf, [pos], d16 - base, mask=m)
                off1 = off0 + cum[LANES - 1]
                off_ref[0] = off1

                @pl.when(off1 >= FLUSH)
                def _():
                    fc = off_ref[1]

                    @pl.when(fc % 2 == 0)
                    def _():
                        _flush(0, fc)

                    @pl.when(fc % 2 == 1)
                    def _():
                        _flush(1, fc)
                    src_buf[pl.ds(0, LANES)] = src_buf[pl.ds(FLUSH, LANES)]
                    dst_buf[pl.ds(0, LANES)] = dst_buf[pl.ds(FLUSH, LANES)]
                    off_ref[0] = off1 - FLUSH
                    off_ref[1] = fc + 1

        _blk_start(0, bs0, bd0, sembs0, sembd0)

        @pl.loop(0, NBLK // 2)
        def _(t):
            b0 = 2 * t
            _blk_wait(b0, bs0, bd0, sembs0, sembd0)
            _blk_start(b0 + 1, bs1, bd1, sembs1, sembd1)
            _chunks(bs0, bd0)
            _blk_wait(b0 + 1, bs1, bd1, sembs1, sembd1)

            @pl.when(t < NBLK // 2 - 1)
            def _():
                _blk_start(b0 + 2, bs0, bd0, sembs0, sembd0)
            _chunks(bs1, bd1)

        fcf = off_ref[1]

        def _wait_adds(q):
            sd, dd, rows, semg, sema, semc = BUFS[q]
            pltpu.make_async_copy(rows, acc.at[dd], sema).wait()
            pltpu.make_async_copy(ones_v, cnt_sh.at[dd], semc).wait()

        def _last_scatter(q):
            sd, dd, rows, semg, sema, semc = BUFS[q]
            pltpu.make_async_copy(x_hbm.at[sd], rows, semg).wait()
            pltpu.sync_copy(rows, acc.at[dd], add=True)
            pltpu.sync_copy(ones_v, cnt_sh.at[dd], add=True)

        @pl.when((fcf >= 2) & (fcf % 2 == 0))
        def _():
            _wait_adds(0)

        @pl.when((fcf >= 2) & (fcf % 2 == 1))
        def _():
            _wait_adds(1)

        @pl.when((fcf >= 1) & (fcf % 2 == 1))
        def _():
            _last_scatter(0)

        @pl.when((fcf >= 1) & (fcf % 2 == 0))
        def _():
            _last_scatter(1)

        offf = off_ref[0]

        @pl.when(offf > 0)
        def _():
            for k in range(FLUSH // LANES):
                sl = pl.ds(k * LANES, LANES)
                posv = lax.iota(jnp.int32, LANES) + (k * LANES)
                keep = posv < offf
                sdma0[sl] = src_buf[sl]
                ddma0[sl] = jnp.where(
                    keep, dst_buf[sl], jnp.full((LANES,), DUMMY, jnp.int32))
            pltpu.async_copy(x_hbm.at[sdma0], rows0, semg0).wait()
            pltpu.sync_copy(rows0, acc.at[ddma0], add=True)
            pltpu.sync_copy(ones_v, cnt_sh.at[ddma0], add=True)

        plsc.subcore_barrier()

        @pl.when(s < NSUB - 1)
        def _():
            pltpu.sync_copy(acc.at[pl.ds(row0, ROW_CHUNK)],
                            sum_hbm.at[pl.ds(base + row0, ROW_CHUNK)])

        @pl.when(s == NSUB - 1)
        def _():
            pltpu.sync_copy(acc.at[pl.ds(row0, ROW_LAST)],
                            sum_hbm.at[pl.ds(base + row0, ROW_LAST)])

        @pl.when(s < NSUB - 1)
        def _():
            pltpu.sync_copy(cnt_sh.at[pl.ds(s * CNT_CHUNK, CNT_CHUNK)], cbuf)
            pltpu.sync_copy(cbuf,
                            cnt_hbm.at[pl.ds(base + s * CNT_CHUNK, CNT_CHUNK)])

        @pl.when(s == NSUB - 1)
        def _():
            pltpu.sync_copy(cnt_sh.at[pl.ds(s * CNT_CHUNK, CNT_LAST)],
                            cbuf.at[pl.ds(0, CNT_LAST)])
            pltpu.sync_copy(cbuf.at[pl.ds(0, CNT_LAST)],
                            cnt_hbm.at[pl.ds(base + s * CNT_CHUNK, CNT_LAST)])

        plsc.subcore_barrier()


def _sc_segsum(x, src, dst):
    mesh = plsc.VectorSubcoreMesh(core_axis_name="c", subcore_axis_name="s")
    f = pl.kernel(
        _segsum_body,
        out_type=(jax.ShapeDtypeStruct((NPAD, D), jnp.float32),
                  jax.ShapeDtypeStruct((NPAD,), jnp.float32)),
        mesh=mesh,
        compiler_params=dataclasses.replace(
            pltpu.CompilerParams(), needs_layout_passes=False),
        scratch_types=[
            pltpu.VMEM((FLUSH, D), jnp.float32),
            pltpu.VMEM((FLUSH, D), jnp.float32),
            pltpu.VMEM((BLKE,), jnp.int32),
            pltpu.VMEM((BLKE,), jnp.int32),
            pltpu.VMEM((BLKE,), jnp.int32),
            pltpu.VMEM((BLKE,), jnp.int32),
            pltpu.VMEM((CAP,), jnp.int32),
            pltpu.VMEM((CAP,), jnp.int32),
            pltpu.VMEM((FLUSH,), jnp.int32),
            pltpu.VMEM((FLUSH,), jnp.int32),
            pltpu.VMEM((FLUSH,), jnp.int32),
            pltpu.VMEM((FLUSH,), jnp.int32),
            pltpu.VMEM((CNT_CHUNK,), jnp.float32),
            pltpu.VMEM((FLUSH,), jnp.float32),
            pltpu.VMEM((CNT_CHUNK,), jnp.float32),
            pltpu.SMEM((2,), jnp.int32),
            pltpu.VMEM_SHARED((ACC_ROWS, D), jnp.float32),
            pltpu.VMEM_SHARED((ACC_ROWS,), jnp.float32),
            pltpu.SemaphoreType.DMA,
            pltpu.SemaphoreType.DMA,
            pltpu.SemaphoreType.DMA,
            pltpu.SemaphoreType.DMA,
            pltpu.SemaphoreType.DMA,
            pltpu.SemaphoreType.DMA,
            pltpu.SemaphoreType.DMA,
            pltpu.SemaphoreType.DMA,
            pltpu.SemaphoreType.DMA,
            pltpu.SemaphoreType.DMA,
        ],
    )
    return f(x, src, dst)


_BLKR = 5000


def _linr_body(x_ref, wr_ref, o_ref):
    o_ref[...] = lax.dot_general(x_ref[...], wr_ref[...],
                                 (((1,), (0,)), ((), ())),
                                 precision=lax.Precision.DEFAULT,
                                 preferred_element_type=jnp.float32)


def _tc_linr(x_dst, wr):
    return pl.pallas_call(
        _linr_body,
        grid=(N // _BLKR,),
        in_specs=[
            pl.BlockSpec((_BLKR, D), lambda i: (i, 0)),
            pl.BlockSpec((D, D), lambda i: (0, 0)),
        ],
        out_specs=pl.BlockSpec((_BLKR, D), lambda i: (i, 0)),
        out_shape=jax.ShapeDtypeStruct((N, D), jnp.float32),
    )(x_dst, wr)


_BLKF = 5000


def _finish_body(sum_ref, rcp_ref, lr_ref, wl_ref, b_ref, o_ref):
    agg = sum_ref[...] * rcp_ref[...]
    y = lax.dot_general(agg, wl_ref[...], (((1,), (0,)), ((), ())),
                        precision=lax.Precision.DEFAULT,
                        preferred_element_type=jnp.float32)
    o_ref[...] = jnp.maximum(y + lr_ref[...] + b_ref[...], 0.0)


def _tc_finish(sum_pad, rcp2, linr, wl, bb):
    return pl.pallas_call(
        _finish_body,
        grid=(N // _BLKF,),
        in_specs=[
            pl.BlockSpec((_BLKF, D), lambda i: (i, 0)),
            pl.BlockSpec((_BLKF, 1), lambda i: (i, 0)),
            pl.BlockSpec((_BLKF, D), lambda i: (i, 0)),
            pl.BlockSpec((D, D), lambda i: (0, 0)),
            pl.BlockSpec((1, D), lambda i: (0, 0)),
        ],
        out_specs=pl.BlockSpec((_BLKF, D), lambda i: (i, 0)),
        out_shape=jax.ShapeDtypeStruct((N, D), jnp.float32),
    )(sum_pad, rcp2, linr, wl, bb)


def _pad_edges(edge):
    src = jnp.concatenate(
        [edge[0].astype(jnp.int32), jnp.zeros((EPAD - E_RAW,), jnp.int32)])
    dst = jnp.concatenate(
        [edge[1].astype(jnp.int32),
         jnp.full((EPAD - E_RAW,), SENTINEL, jnp.int32)])
    return src, dst


def kernel(x_user, x_event, edge_e2u, edge_u2e,
           Wl_u0, bl_u0, Wr_u0, gamma_u0, beta_u0,
           Wl_e0, bl_e0, Wr_e0, gamma_e0, beta_e0,
           Wl_u1, bl_u1, Wr_u1, gamma_u1, beta_u1,
           Wl_e1, bl_e1, Wr_e1, gamma_e1, beta_e1):
    se2u, de2u = _pad_edges(edge_e2u)
    su2e, du2e = _pad_edges(edge_u2e)
    params = {
        "u": [(Wl_u0, bl_u0, Wr_u0, gamma_u0, beta_u0),
              (Wl_u1, bl_u1, Wr_u1, gamma_u1, beta_u1)],
        "e": [(Wl_e0, bl_e0, Wr_e0, gamma_e0, beta_e0),
              (Wl_e1, bl_e1, Wr_e1, gamma_e1, beta_e1)],
    }

    def _prep(Wl, bl, Wr, gamma, beta):
        scale = gamma * _BN_SCALE
        return (Wl.T * scale[None, :], Wr.T * scale[None, :],
                (bl * scale + beta).reshape(1, D))

    xu, xe = x_user, x_event
    rcp_u = rcp_e = None
    for i in range(2):
        wl, wr, bb = _prep(*params["u"][i])
        lr = _tc_linr(xu, wr)
        su, cu = _sc_segsum(xe, se2u, de2u)
        if rcp_u is None:
            rcp_u = (1.0 / jnp.maximum(cu, 1.0)).reshape(-1, 1)
        xu = _tc_finish(su, rcp_u, lr, wl, bb)
        wl, wr, bb = _prep(*params["e"][i])
        lr = _tc_linr(xe, wr)
        se, ce = _sc_segsum(xu, su2e, du2e)
        if rcp_e is None:
            rcp_e = (1.0 / jnp.maximum(ce, 1.0)).reshape(-1, 1)
        xe = _tc_finish(se, rcp_e, lr, wl, bb)
    return (xu, xe)


# speedup vs baseline: 13.6187x; 1.0063x over previous
import dataclasses
import math

import jax
import jax.numpy as jnp
from jax import lax
from jax.experimental import pallas as pl
from jax.experimental.pallas import tpu as pltpu
from jax.experimental.pallas import tpu_sc as plsc

D = 128
N = 50000
E_RAW = 600000
LANES = 16
NSUB = 16

BLKE = 1184
NBLK = 32
EPAD = BLKE * NSUB * NBLK
NPASS = 3
FLUSH = 208
ZROWS = 128
CAP = 224
REG = 8344
NPAD = 2 * NPASS * REG
ACC_ROWS = 8352
DUMMY = REG
SENTINEL = 1 << 30
ROW_CHUNK = 528
ROW_LAST = REG - 15 * ROW_CHUNK
CNT_CHUNK = 528
CNT_LAST = REG - 15 * CNT_CHUNK

_BN_SCALE = 1.0 / math.sqrt(1.0 + 1e-5)


def _segsum_body(x_hbm, src_hbm, dst_hbm, sum_hbm, cnt_hbm,
                 rows0, rows1, bs0, bd0, bs1, bd1, src_buf, dst_buf,
                 sdma0, ddma0, sdma1, ddma1, zbuf, ones_v, cbuf, off_ref,
                 acc, cnt_sh,
                 semg0, semg1, sema0, sema1, semc0, semc1,
                 sembs0, sembd0, sembs1, sembd1):
    c = lax.axis_index("c")
    s = lax.axis_index("s")
    z16f = jnp.zeros((LANES,), jnp.float32)
    z16i = jnp.zeros((LANES,), jnp.int32)
    one16 = jnp.ones((LANES,), jnp.float32)
    row0 = s * ROW_CHUNK

    @pl.loop(0, CNT_CHUNK // LANES)
    def _(i):
        zbuf[pl.ds(i * LANES, LANES)] = z16f

    for k in range(FLUSH // LANES):
        ones_v[pl.ds(k * LANES, LANES)] = one16

    BUFS = ((sdma0, ddma0, rows0, semg0, sema0, semc0),
            (sdma1, ddma1, rows1, semg1, sema1, semc1))

    for p in range(NPASS):
        base = (2 * p + c) * REG

        @pl.loop(0, ZROWS)
        def _(r):
            for k in range(D // LANES):
                rows0[r, pl.ds(k * LANES, LANES)] = z16f

        zstage = rows0.at[pl.ds(0, ZROWS)]
        for k in range(3):
            pltpu.sync_copy(zstage, acc.at[pl.ds(row0 + k * ZROWS, ZROWS)])

        @pl.when(s < NSUB - 1)
        def _():
            pltpu.sync_copy(zstage, acc.at[pl.ds(row0 + 3 * ZROWS, ZROWS)])
            pltpu.sync_copy(rows0.at[pl.ds(0, ROW_CHUNK - 4 * ZROWS)],
                            acc.at[pl.ds(row0 + 4 * ZROWS,
                                         ROW_CHUNK - 4 * ZROWS)])

        @pl.when(s == NSUB - 1)
        def _():
            pltpu.sync_copy(rows0.at[pl.ds(0, ROW_LAST - 3 * ZROWS)],
                            acc.at[pl.ds(row0 + 3 * ZROWS,
                                         ROW_LAST - 3 * ZROWS)])

        @pl.when(s < NSUB - 1)
        def _():
            pltpu.sync_copy(zbuf, cnt_sh.at[pl.ds(s * CNT_CHUNK, CNT_CHUNK)])

        @pl.when(s == NSUB - 1)
        def _():
            pltpu.sync_copy(zbuf.at[pl.ds(0, CNT_LAST)],
                            cnt_sh.at[pl.ds(s * CNT_CHUNK, CNT_LAST)])

        for k in range(CAP // LANES):
            src_buf[pl.ds(k * LANES, LANES)] = z16i
            dst_buf[pl.ds(k * LANES, LANES)] = z16i
        off_ref[0] = 0
        off_ref[1] = 0
        plsc.subcore_barrier()

        def _blk_start(b, bs, bd, ss, sd):
            e0 = (b * NSUB + s) * BLKE
            pltpu.async_copy(src_hbm.at[pl.ds(e0, BLKE)], bs, ss)
            pltpu.async_copy(dst_hbm.at[pl.ds(e0, BLKE)], bd, sd)

        def _blk_wait(b, bs, bd, ss, sd):
            e0 = (b * NSUB + s) * BLKE
            pltpu.make_async_copy(src_hbm.at[pl.ds(e0, BLKE)], bs, ss).wait()
            pltpu.make_async_copy(dst_hbm.at[pl.ds(e0, BLKE)], bd, sd).wait()

        def _flush(q, fc):
            sd, dd, rows, semg, sema, semc = BUFS[q]
            osd, odd, orows, osemg, osema, osemc = BUFS[1 - q]

            @pl.when(fc >= 2)
            def _():
                pltpu.make_async_copy(rows, acc.at[dd], sema).wait()
                pltpu.make_async_copy(ones_v, cnt_sh.at[dd], semc).wait()
            for k in range(FLUSH // LANES):
                sl = pl.ds(k * LANES, LANES)
                sd[sl] = src_buf[sl]
                dd[sl] = dst_buf[sl]
            pltpu.async_copy(x_hbm.at[sd], rows, semg)

            @pl.when(fc >= 1)
            def _():
                pltpu.make_async_copy(x_hbm.at[osd], orows, osemg).wait()
                pltpu.async_copy(orows, acc.at[odd], osema, add=True)
                pltpu.async_copy(ones_v, cnt_sh.at[odd], osemc, add=True)

        def _chunks(bs, bd):
            @pl.loop(0, BLKE // LANES, unroll=2)
            def _(k):
                sl = pl.ds(k * LANES, LANES)
                d16 = bd[sl]
                s16 = bs[sl]
                m = (d16 >= base) & (d16 < base + REG)
                mi = m.astype(jnp.int32)
                off0 = off_ref[0]
                cum = jnp.cumsum(mi)
                pos = cum - mi + off0
                plsc.store_scatter(src_buf, [pos], s16, mask=m)
                plsc.store_scatter(dst_buf, [pos], d16 - base, mask=m)
                off1 = off0 + cum[LANES - 1]
                off_ref[0] = off1

                @pl.when(off1 >= FLUSH)
                def _():
                    fc = off_ref[1]

                    @pl.when(fc % 2 == 0)
                    def _():
                        _flush(0, fc)

                    @pl.when(fc % 2 == 1)
                    def _():
                        _flush(1, fc)
                    src_buf[pl.ds(0, LANES)] = src_buf[pl.ds(FLUSH, LANES)]
                    dst_buf[pl.ds(0, LANES)] = dst_buf[pl.ds(FLUSH, LANES)]
                    off_ref[0] = off1 - FLUSH
                    off_ref[1] = fc + 1

        _blk_start(0, bs0, bd0, sembs0, sembd0)

        @pl.loop(0, NBLK // 2)
        def _(t):
            b0 = 2 * t
            _blk_wait(b0, bs0, bd0, sembs0, sembd0)
            _blk_start(b0 + 1, bs1, bd1, sembs1, sembd1)
            _chunks(bs0, bd0)
            _blk_wait(b0 + 1, bs1, bd1, sembs1, sembd1)

            @pl.when(t < NBLK // 2 - 1)
            def _():
                _blk_start(b0 + 2, bs0, bd0, sembs0, sembd0)
            _chunks(bs1, bd1)

        fcf = off_ref[1]

        def _wait_adds(q):
            sd, dd, rows, semg, sema, semc = BUFS[q]
            pltpu.make_async_copy(rows, acc.at[dd], sema).wait()
            pltpu.make_async_copy(ones_v, cnt_sh.at[dd], semc).wait()

        def _last_scatter(q):
            sd, dd, rows, semg, sema, semc = BUFS[q]
            pltpu.make_async_copy(x_hbm.at[sd], rows, semg).wait()
            pltpu.sync_copy(rows, acc.at[dd], add=True)
            pltpu.sync_copy(ones_v, cnt_sh.at[dd], add=True)

        @pl.when((fcf >= 2) & (fcf % 2 == 0))
        def _():
            _wait_adds(0)

        @pl.when((fcf >= 2) & (fcf % 2 == 1))
        def _():
            _wait_adds(1)

        @pl.when((fcf >= 1) & (fcf % 2 == 1))
        def _():
            _last_scatter(0)

        @pl.when((fcf >= 1) & (fcf % 2 == 0))
        def _():
            _last_scatter(1)

        offf = off_ref[0]

        @pl.when(offf > 0)
        def _():
            for k in range(FLUSH // LANES):
                sl = pl.ds(k * LANES, LANES)
                posv = lax.iota(jnp.int32, LANES) + (k * LANES)
                keep = posv < offf
                sdma0[sl] = src_buf[sl]
                ddma0[sl] = jnp.where(
                    keep, dst_buf[sl], jnp.full((LANES,), DUMMY, jnp.int32))
            pltpu.async_copy(x_hbm.at[sdma0], rows0, semg0).wait()
            pltpu.sync_copy(rows0, acc.at[ddma0], add=True)
            pltpu.sync_copy(ones_v, cnt_sh.at[ddma0], add=True)

        plsc.subcore_barrier()

        @pl.when(s < NSUB - 1)
        def _():
            pltpu.sync_copy(acc.at[pl.ds(row0, ROW_CHUNK)],
                            sum_hbm.at[pl.ds(base + row0, ROW_CHUNK)])

        @pl.when(s == NSUB - 1)
        def _():
            pltpu.sync_copy(acc.at[pl.ds(row0, ROW_LAST)],
                            sum_hbm.at[pl.ds(base + row0, ROW_LAST)])

        @pl.when(s < NSUB - 1)
        def _():
            pltpu.sync_copy(cnt_sh.at[pl.ds(s * CNT_CHUNK, CNT_CHUNK)], cbuf)
            pltpu.sync_copy(cbuf,
                            cnt_hbm.at[pl.ds(base + s * CNT_CHUNK, CNT_CHUNK)])

        @pl.when(s == NSUB - 1)
        def _():
            pltpu.sync_copy(cnt_sh.at[pl.ds(s * CNT_CHUNK, CNT_LAST)],
                            cbuf.at[pl.ds(0, CNT_LAST)])
            pltpu.sync_copy(cbuf.at[pl.ds(0, CNT_LAST)],
                            cnt_hbm.at[pl.ds(base + s * CNT_CHUNK, CNT_LAST)])

        plsc.subcore_barrier()


def _sc_segsum(x, src, dst):
    mesh = plsc.VectorSubcoreMesh(core_axis_name="c", subcore_axis_name="s")
    f = pl.kernel(
        _segsum_body,
        out_type=(jax.ShapeDtypeStruct((NPAD, D), jnp.float32),
                  jax.ShapeDtypeStruct((NPAD,), jnp.float32)),
        mesh=mesh,
        compiler_params=dataclasses.replace(
            pltpu.CompilerParams(), needs_layout_passes=False),
        scratch_types=[
            pltpu.VMEM((FLUSH, D), jnp.float32),
            pltpu.VMEM((FLUSH, D), jnp.float32),
            pltpu.VMEM((BLKE,), jnp.int32),
            pltpu.VMEM((BLKE,), jnp.int32),
            pltpu.VMEM((BLKE,), jnp.int32),
            pltpu.VMEM((BLKE,), jnp.int32),
            pltpu.VMEM((CAP,), jnp.int32),
            pltpu.VMEM((CAP,), jnp.int32),
            pltpu.VMEM((FLUSH,), jnp.int32),
            pltpu.VMEM((FLUSH,), jnp.int32),
            pltpu.VMEM((FLUSH,), jnp.int32),
            pltpu.VMEM((FLUSH,), jnp.int32),
            pltpu.VMEM((CNT_CHUNK,), jnp.float32),
            pltpu.VMEM((FLUSH,), jnp.float32),
            pltpu.VMEM((CNT_CHUNK,), jnp.float32),
            pltpu.SMEM((2,), jnp.int32),
            pltpu.VMEM_SHARED((ACC_ROWS, D), jnp.float32),
            pltpu.VMEM_SHARED((ACC_ROWS,), jnp.float32),
            pltpu.SemaphoreType.DMA,
            pltpu.SemaphoreType.DMA,
            pltpu.SemaphoreType.DMA,
            pltpu.SemaphoreType.DMA,
            pltpu.SemaphoreType.DMA,
            pltpu.SemaphoreType.DMA,
            pltpu.SemaphoreType.DMA,
            pltpu.SemaphoreType.DMA,
            pltpu.SemaphoreType.DMA,
            pltpu.SemaphoreType.DMA,
        ],
    )
    return f(x, src, dst)


_BLKR = 5000


def _linr_body(x_ref, wr_ref, o_ref):
    o_ref[...] = lax.dot_general(x_ref[...], wr_ref[...],
                                 (((1,), (0,)), ((), ())),
                                 precision=lax.Precision.DEFAULT,
                                 preferred_element_type=jnp.float32)


def _tc_linr(x_dst, wr):
    return pl.pallas_call(
        _linr_body,
        grid=(N // _BLKR,),
        in_specs=[
            pl.BlockSpec((_BLKR, D), lambda i: (i, 0)),
            pl.BlockSpec((D, D), lambda i: (0, 0)),
        ],
        out_specs=pl.BlockSpec((_BLKR, D), lambda i: (i, 0)),
        out_shape=jax.ShapeDtypeStruct((N, D), jnp.float32),
    )(x_dst, wr)


_BLKF = 5000


def _finish_body(sum_ref, rcp_ref, lr_ref, wl_ref, b_ref, o_ref):
    agg = sum_ref[...] * rcp_ref[...]
    y = lax.dot_general(agg, wl_ref[...], (((1,), (0,)), ((), ())),
                        precision=lax.Precision.DEFAULT,
                        preferred_element_type=jnp.float32)
    o_ref[...] = jnp.maximum(y + lr_ref[...] + b_ref[...], 0.0)


def _tc_finish(sum_pad, rcp2, linr, wl, bb):
    return pl.pallas_call(
        _finish_body,
        grid=(N // _BLKF,),
        in_specs=[
            pl.BlockSpec((_BLKF, D), lambda i: (i, 0)),
            pl.BlockSpec((_BLKF, 1), lambda i: (i, 0)),
            pl.BlockSpec((_BLKF, D), lambda i: (i, 0)),
            pl.BlockSpec((D, D), lambda i: (0, 0)),
            pl.BlockSpec((1, D), lambda i: (0, 0)),
        ],
        out_specs=pl.BlockSpec((_BLKF, D), lambda i: (i, 0)),
        out_shape=jax.ShapeDtypeStruct((N, D), jnp.float32),
    )(sum_pad, rcp2, linr, wl, bb)


def _pad_edges(edge):
    src = jnp.concatenate(
        [edge[0].astype(jnp.int32), jnp.zeros((EPAD - E_RAW,), jnp.int32)])
    dst = jnp.concatenate(
        [edge[1].astype(jnp.int32),
         jnp.full((EPAD - E_RAW,), SENTINEL, jnp.int32)])
    return src, dst


def kernel(x_user, x_event, edge_e2u, edge_u2e,
           Wl_u0, bl_u0, Wr_u0, gamma_u0, beta_u0,
           Wl_e0, bl_e0, Wr_e0, gamma_e0, beta_e0,
           Wl_u1, bl_u1, Wr_u1, gamma_u1, beta_u1,
           Wl_e1, bl_e1, Wr_e1, gamma_e1, beta_e1):
    se2u, de2u = _pad_edges(edge_e2u)
    su2e, du2e = _pad_edges(edge_u2e)
    params = {
        "u": [(Wl_u0, bl_u0, Wr_u0, gamma_u0, beta_u0),
              (Wl_u1, bl_u1, Wr_u1, gamma_u1, beta_u1)],
        "e": [(Wl_e0, bl_e0, Wr_e0, gamma_e0, beta_e0),
              (Wl_e1, bl_e1, Wr_e1, gamma_e1, beta_e1)],
    }

    def _prep(Wl, bl, Wr, gamma, beta):
        scale = gamma * _BN_SCALE
        return (Wl.T * scale[None, :], Wr.T * scale[None, :],
                (bl * scale + beta).reshape(1, D))

    xu, xe = x_user, x_event
    rcp_u = rcp_e = None
    for i in range(2):
        wl, wr, bb = _prep(*params["u"][i])
        lr = _tc_linr(xu, wr)
        su, cu = _sc_segsum(xe, se2u, de2u)
        if rcp_u is None:
            rcp_u = (1.0 / jnp.maximum(cu, 1.0)).reshape(-1, 1)
        xu = _tc_finish(su, rcp_u, lr, wl, bb)
        wl, wr, bb = _prep(*params["e"][i])
        lr = _tc_linr(xe, wr)
        se, ce = _sc_segsum(xu, su2e, du2e)
        if rcp_e is None:
            rcp_e = (1.0 / jnp.maximum(ce, 1.0)).reshape(-1, 1)
        xe = _tc_finish(se, rcp_e, lr, wl, bb)
    return (xu, xe)
